# Initial kernel scaffold; baseline (speedup 1.0000x reference)
import jax
import jax.numpy as jnp
from jax.experimental import pallas as pl


def kernel(x, edge_index, batch, target1, target2, params):
    raise NotImplementedError("write your pallas kernel here")


# speedup vs baseline: 5.6036x; 5.6036x over previous
import jax
import jax.numpy as jnp
from jax import lax
from jax.experimental import pallas as pl
from jax.experimental.pallas import tpu as pltpu
from jax.experimental.pallas import tpu_sc as plsc

N_GRAPHS = 128
N_NODES = 50000
DIM = 32
N_EDGES = 800000

_NC, _NS = 2, 16
_CH = 128
_CHUNKS = N_EDGES // _CH
_CPC = _CHUNKS // _NC
_RPS = 3120
_ZR = 120
_NZ = _RPS // _ZR
_REM0 = _NS * _RPS
_REM = N_NODES - _REM0

_sc_mesh = plsc.VectorSubcoreMesh(
    core_axis_name="c", subcore_axis_name="s", num_cores=_NC, num_subcores=_NS)


def _segsum_body(y_hbm, src_hbm, dst_hbm, out_hbm, srcv, dstv, rows, zbuf, acc,
                 sem):
    cid = lax.axis_index("c")
    sid = lax.axis_index("s")

    zv = jnp.zeros((16,), jnp.float32)

    def zb(i, carry):
        zbuf[i, pl.ds(0, 16)] = zv
        zbuf[i, pl.ds(16, 16)] = zv
        return carry

    lax.fori_loop(0, _ZR, zb, 0)

    def zacc(i, carry):
        pltpu.sync_copy(zbuf, acc.at[pl.ds(sid * _RPS + i * _ZR, _ZR)])
        return carry

    lax.fori_loop(0, _NZ, zacc, 0)

    @pl.when(sid == 0)
    def _():
        pltpu.sync_copy(zbuf.at[pl.ds(0, _REM)], acc.at[pl.ds(_REM0, _REM)])

    plsc.subcore_barrier()

    nbase = _CPC // _NS
    extra = _CPC - nbase * _NS
    count = nbase + jnp.where(sid < extra, 1, 0)

    def edge_step(k, carry):
        off = (cid * _CPC + k * _NS + sid) * _CH
        pltpu.sync_copy(src_hbm.at[pl.ds(off, _CH)], srcv)
        pltpu.sync_copy(dst_hbm.at[pl.ds(off, _CH)], dstv)
        pltpu.async_copy(y_hbm.at[srcv], rows, sem).wait()
        pltpu.sync_copy(rows, acc.at[dstv], add=True)
        return carry

    lax.fori_loop(0, count, edge_step, 0)
    plsc.subcore_barrier()

    def flush(i, carry):
        r0 = sid * _RPS + i * _ZR
        pltpu.sync_copy(acc.at[pl.ds(r0, _ZR)],
                        out_hbm.at[pl.ds(cid * N_NODES + r0, _ZR)])
        return carry

    lax.fori_loop(0, _NZ, flush, 0)

    @pl.when(sid == 0)
    def _():
        pltpu.sync_copy(acc.at[pl.ds(_REM0, _REM)],
                        out_hbm.at[pl.ds(cid * N_NODES + _REM0, _REM)])


_segsum_call = pl.kernel(
    _segsum_body,
    out_type=jax.ShapeDtypeStruct((_NC * N_NODES, DIM), jnp.float32),
    mesh=_sc_mesh,
    scratch_types=[
        pltpu.VMEM((_CH,), jnp.int32),
        pltpu.VMEM((_CH,), jnp.int32),
        pltpu.VMEM((_CH, DIM), jnp.float32),
        pltpu.VMEM((_ZR, DIM), jnp.float32),
        pltpu.VMEM_SHARED((N_NODES, DIM), jnp.float32),
        pltpu.SemaphoreType.DMA,
    ],
    compiler_params=pltpu.CompilerParams(use_tc_tiling_on_sc=False),
)


def _segsum(y, src, dst):
    s = _segsum_call(y, src, dst)
    return s[:N_NODES], s[N_NODES:]


_BLK = 1000
_NBLK = N_NODES // _BLK


def _matmul_body(x_ref, w_ref, o_ref):
    o_ref[...] = jnp.dot(x_ref[...], w_ref[...],
                         preferred_element_type=jnp.float32)


def _node_matmul(x, w):
    fin = x.shape[1]
    return pl.pallas_call(
        _matmul_body,
        grid=(_NBLK,),
        in_specs=[pl.BlockSpec((_BLK, fin), lambda i: (i, 0)),
                  pl.BlockSpec((fin, DIM), lambda i: (0, 0))],
        out_specs=pl.BlockSpec((_BLK, DIM), lambda i: (i, 0)),
        out_shape=jax.ShapeDtypeStruct((N_NODES, DIM), jnp.float32),
    )(x, w)


def _stats_accum(m, sum_ref, ssq_ref):
    ones = jnp.ones((8, m.shape[0]), jnp.float32)
    ps = jnp.dot(ones, m, preferred_element_type=jnp.float32)
    pq = jnp.dot(ones, m * m, preferred_element_type=jnp.float32)

    @pl.when(pl.program_id(0) == 0)
    def _():
        sum_ref[...] = ps
        ssq_ref[...] = pq

    @pl.when(pl.program_id(0) != 0)
    def _():
        sum_ref[...] += ps
        ssq_ref[...] += pq


def _gin0_body(y_ref, s0_ref, s1_ref, b1_ref, w2_ref, b2_ref,
               m_ref, sum_ref, ssq_ref):
    u = jnp.maximum(y_ref[...] + s0_ref[...] + s1_ref[...] + b1_ref[...], 0.0)
    m = jnp.maximum(
        jnp.dot(u, w2_ref[...], preferred_element_type=jnp.float32)
        + b2_ref[...], 0.0)
    m_ref[...] = m
    _stats_accum(m, sum_ref, ssq_ref)


def _gin_body(h_ref, s0_ref, s1_ref, w1_ref, b1_ref, w2_ref, b2_ref,
              m_ref, sum_ref, ssq_ref):
    t = h_ref[...] + s0_ref[...] + s1_ref[...]
    u = jnp.maximum(
        jnp.dot(t, w1_ref[...], preferred_element_type=jnp.float32)
        + b1_ref[...], 0.0)
    m = jnp.maximum(
        jnp.dot(u, w2_ref[...], preferred_element_type=jnp.float32)
        + b2_ref[...], 0.0)
    m_ref[...] = m
    _stats_accum(m, sum_ref, ssq_ref)


_nodes_spec = pl.BlockSpec((_BLK, DIM), lambda i: (i, 0))
_w_spec = pl.BlockSpec((DIM, DIM), lambda i: (0, 0))
_b_spec = pl.BlockSpec((1, DIM), lambda i: (0, 0))
_stat_spec = pl.BlockSpec((8, DIM), lambda i: (0, 0))
_mlp_out_shapes = (
    jax.ShapeDtypeStruct((N_NODES, DIM), jnp.float32),
    jax.ShapeDtypeStruct((8, DIM), jnp.float32),
    jax.ShapeDtypeStruct((8, DIM), jnp.float32),
)


def _gin_mlp0(y, s0, s1, b1, w2, b2):
    return pl.pallas_call(
        _gin0_body,
        grid=(_NBLK,),
        in_specs=[_nodes_spec, _nodes_spec, _nodes_spec,
                  _b_spec, _w_spec, _b_spec],
        out_specs=(_nodes_spec, _stat_spec, _stat_spec),
        out_shape=_mlp_out_shapes,
    )(y, s0, s1, b1, w2, b2)


def _gin_mlp(h, s0, s1, w1, b1, w2, b2):
    return pl.pallas_call(
        _gin_body,
        grid=(_NBLK,),
        in_specs=[_nodes_spec, _nodes_spec, _nodes_spec,
                  _w_spec, _b_spec, _w_spec, _b_spec],
        out_specs=(_nodes_spec, _stat_spec, _stat_spec),
        out_shape=_mlp_out_shapes,
    )(h, s0, s1, w1, b1, w2, b2)


def _bn_scale(sum_v, ssq_v, g_v, b_v):
    s = jnp.sum(sum_v, axis=0, keepdims=True)
    q = jnp.sum(ssq_v, axis=0, keepdims=True)
    mu = s * (1.0 / N_NODES)
    var = q * (1.0 / N_NODES) - mu * mu
    a = g_v * lax.rsqrt(var + 1e-5)
    c = b_v - mu * a
    return a, c


def _bn_body(m_ref, sum_ref, ssq_ref, g_ref, b_ref, h_ref):
    a, c = _bn_scale(sum_ref[...], ssq_ref[...], g_ref[...], b_ref[...])
    h_ref[...] = m_ref[...] * a + c


def _bn_apply(m, s8, q8, g, b):
    return pl.pallas_call(
        _bn_body,
        grid=(_NBLK,),
        in_specs=[_nodes_spec, _stat_spec, _stat_spec, _b_spec, _b_spec],
        out_specs=_nodes_spec,
        out_shape=jax.ShapeDtypeStruct((N_NODES, DIM), jnp.float32),
    )(m, s8, q8, g, b)


def _bn_pool_body(m_ref, sum_ref, ssq_ref, g_ref, b_ref, bt_ref, p_ref):
    a, c = _bn_scale(sum_ref[...], ssq_ref[...], g_ref[...], b_ref[...])
    h = m_ref[...] * a + c
    brow = bt_ref[...].reshape(1, _BLK)
    oh = (lax.broadcasted_iota(jnp.int32, (N_GRAPHS, _BLK), 0)
          == brow).astype(jnp.float32)
    pp = jnp.dot(oh, h, preferred_element_type=jnp.float32)

    @pl.when(pl.program_id(0) == 0)
    def _():
        p_ref[...] = pp

    @pl.when(pl.program_id(0) != 0)
    def _():
        p_ref[...] += pp


def _bn_pool(m, s8, q8, g, b, batch3):
    return pl.pallas_call(
        _bn_pool_body,
        grid=(_NBLK,),
        in_specs=[_nodes_spec, _stat_spec, _stat_spec, _b_spec, _b_spec,
                  pl.BlockSpec((1, 1, _BLK), lambda i: (i, 0, 0))],
        out_specs=pl.BlockSpec((N_GRAPHS, DIM), lambda i: (0, 0)),
        out_shape=jax.ShapeDtypeStruct((N_GRAPHS, DIM), jnp.float32),
    )(m, s8, q8, g, b, batch3)


def _heads_body(t2_ref, t1_ref, w2s_ref, a2t_ref, b2c_ref, w1s_ref, b1c_ref,
                c2_ref, c1_ref):
    t2 = t2_ref[...].reshape(1, 1000)
    oh = (lax.broadcasted_iota(jnp.int32, (32, 1000), 0)
          == t2).astype(jnp.float32)
    c2 = jnp.zeros((32, 121), jnp.float32)
    for k in range(8):
        bk = lax.dot_general(w2s_ref[k], oh, (((1,), (1,)), ((), ())),
                             preferred_element_type=jnp.float32)
        c2 = c2 + jnp.dot(bk, a2t_ref[k], preferred_element_type=jnp.float32)
    c2_ref[...] = (c2 + b2c_ref[...]).reshape(1, 32, 121)

    t1 = t1_ref[...].reshape(20, 24)
    c1 = jnp.zeros((32, 17), jnp.float32)
    for k in range(8):
        c1 = c1 + jnp.dot(w1s_ref[k], t1[:, k:k + 17],
                          preferred_element_type=jnp.float32)
    c1_ref[...] = (c1 + b1c_ref[...]).reshape(1, 32, 17)


def _conv_heads(t2r, t1, w2s, a2t, b2c, w1s, b1c):
    return pl.pallas_call(
        _heads_body,
        grid=(N_GRAPHS,),
        in_specs=[
            pl.BlockSpec((1, 1, 1000), lambda i: (i, 0, 0)),
            pl.BlockSpec((1, 20, 24), lambda i: (i, 0, 0)),
            pl.BlockSpec((8, 32, 1000), lambda i: (0, 0, 0)),
            pl.BlockSpec((8, 32, 121), lambda i: (0, 0, 0)),
            pl.BlockSpec((32, 1), lambda i: (0, 0)),
            pl.BlockSpec((8, 32, 20), lambda i: (0, 0, 0)),
            pl.BlockSpec((32, 1), lambda i: (0, 0)),
        ],
        out_specs=(pl.BlockSpec((1, 32, 121), lambda i: (i, 0, 0)),
                   pl.BlockSpec((1, 32, 17), lambda i: (i, 0, 0))),
        out_shape=(jax.ShapeDtypeStruct((N_GRAPHS, 32, 121), jnp.float32),
                   jax.ShapeDtypeStruct((N_GRAPHS, 32, 17), jnp.float32)),
    )(t2r, t1, w2s, a2t, b2c, w1s, b1c)


def _final_body(p_ref, c1f_ref, c2f_ref, wxd_ref, bxd_ref, w1x_ref, b1x_ref,
                w2x_ref, b2x_ref, wa_ref, wb_ref, wc_ref, bf1_ref,
                wf2_ref, bf2_ref, wo_ref, bo_ref, o_ref):
    f32 = jnp.float32
    xd = jnp.maximum(
        jnp.dot(p_ref[...], wxd_ref[...], preferred_element_type=f32)
        + bxd_ref[...], 0.0)
    xt1 = jnp.dot(c1f_ref[...], w1x_ref[...],
                  preferred_element_type=f32) + b1x_ref[...]
    xt2 = jnp.dot(c2f_ref[...], w2x_ref[...],
                  preferred_element_type=f32) + b2x_ref[...]
    z = jnp.maximum(
        jnp.dot(xd, wa_ref[...], preferred_element_type=f32)
        + jnp.dot(xt1, wb_ref[...], preferred_element_type=f32)
        + jnp.dot(xt2, wc_ref[...], preferred_element_type=f32)
        + bf1_ref[...], 0.0)
    z2 = jnp.maximum(
        jnp.dot(z, wf2_ref[...], preferred_element_type=f32)
        + bf2_ref[...], 0.0)
    o_ref[...] = jnp.dot(z2, wo_ref[...],
                         preferred_element_type=f32) + bo_ref[...]


def _final_mlp(pooled, c1f, c2f, wxd, bxd, w1x, b1x, w2x, b2x,
               wa, wb, wc, bf1, wf2, bf2, wo, bo):
    return pl.pallas_call(
        _final_body,
        out_shape=jax.ShapeDtypeStruct((N_GRAPHS, 1), jnp.float32),
    )(pooled, c1f, c2f, wxd, bxd, w1x, b1x, w2x, b2x,
      wa, wb, wc, bf1, wf2, bf2, wo, bo)


def kernel(x, edge_index, batch, target1, target2, params):
    p = params
    src = edge_index[0]
    dst = edge_index[1]

    def row(v):
        return v.reshape(1, DIM)

    h = None
    pooled = None
    batch3 = batch.astype(jnp.int32).reshape(_NBLK, 1, _BLK)
    for i in range(5):
        if i == 0:
            y = _node_matmul(x, p['gin0_w1'])
            s0, s1 = _segsum(y, src, dst)
            m, s8, q8 = _gin_mlp0(y, s0, s1, row(p['gin0_b1']),
                                  p['gin0_w2'], row(p['gin0_b2']))
        else:
            s0, s1 = _segsum(h, src, dst)
            m, s8, q8 = _gin_mlp(h, s0, s1, p['gin%d_w1' % i],
                                 row(p['gin%d_b1' % i]), p['gin%d_w2' % i],
                                 row(p['gin%d_b2' % i]))
        if i < 4:
            h = _bn_apply(m, s8, q8, row(p['bn%d_g' % i]), row(p['bn%d_b' % i]))
        else:
            pooled = _bn_pool(m, s8, q8, row(p['bn%d_g' % i]),
                              row(p['bn%d_b' % i]), batch3)

    t2r = target2.astype(jnp.int32).reshape(N_GRAPHS, 1, 1000)
    epad = jnp.concatenate(
        [p['emb'], jnp.zeros((6, 128), jnp.float32)], axis=0)
    a2t = jnp.stack([epad[:, k:k + 121] for k in range(8)])
    w2s = jnp.transpose(p['cxt2_w'], (2, 0, 1))
    w1s = jnp.transpose(p['cxt1_w'], (2, 0, 1))
    c2, c1 = _conv_heads(t2r, target1, w2s, a2t,
                         p['cxt2_b'].reshape(32, 1), w1s,
                         p['cxt1_b'].reshape(32, 1))
    c2f = c2.reshape(N_GRAPHS, 32 * 121)
    c1f = c1.reshape(N_GRAPHS, 32 * 17)

    fw = p['fc1_w']
    out = _final_mlp(
        pooled, c1f, c2f,
        p['fc1_xd_w'], p['fc1_xd_b'].reshape(1, -1),
        p['fc1_xt_w'], p['fc1_xt_b'].reshape(1, -1),
        p['fc2_xt_w'], p['fc2_xt_b'].reshape(1, -1),
        fw[0:128], fw[128:256], fw[256:384], p['fc1_b'].reshape(1, -1),
        p['fc2_w'], p['fc2_b'].reshape(1, -1),
        p['out_w'], p['out_b'].reshape(1, -1))
    return out


# speedup vs baseline: 9.4947x; 1.6944x over previous
import jax
import jax.numpy as jnp
from jax import lax
from jax.experimental import pallas as pl
from jax.experimental.pallas import tpu as pltpu
from jax.experimental.pallas import tpu_sc as plsc

N_GRAPHS = 128
N_NODES = 50000
DIM = 32
N_EDGES = 800000

_NC, _NS = 2, 16
_CH = 128
_CHUNKS = N_EDGES // _CH
_CPC = _CHUNKS // _NC
_RPS = 3120
_ZR = 120
_NZ = _RPS // _ZR
_REM0 = _NS * _RPS
_REM = N_NODES - _REM0

_sc_mesh = plsc.VectorSubcoreMesh(
    core_axis_name="c", subcore_axis_name="s", num_cores=_NC, num_subcores=_NS)


def _segsum_body(y_hbm, src2_hbm, dst2_hbm, out_hbm,
                 sidx, didx, rows, zbuf, acc, semg, sems):
    cid = lax.axis_index("c")
    sid = lax.axis_index("s")

    zv = jnp.zeros((16,), jnp.float32)

    def zb(i, carry):
        zbuf[i, pl.ds(0, 16)] = zv
        zbuf[i, pl.ds(16, 16)] = zv
        return carry

    lax.fori_loop(0, _ZR, zb, 0)

    def zacc(i, carry):
        pltpu.sync_copy(zbuf, acc.at[pl.ds(sid * _RPS + i * _ZR, _ZR)])
        return carry

    lax.fori_loop(0, _NZ, zacc, 0)

    @pl.when(sid == 0)
    def _():
        pltpu.sync_copy(zbuf.at[pl.ds(0, _REM)], acc.at[pl.ds(_REM0, _REM)])

    plsc.subcore_barrier()

    extra = jnp.where(cid == 0, 7, 6)
    nb = 24 + jnp.where(sid < extra, 1, 0)
    core_base = cid * 3128

    def block_step(k, carry):
        bchunk = core_base + (sid + _NS * k) * 8
        pltpu.sync_copy(src2_hbm.at[pl.ds(bchunk, 8)], sidx)
        pltpu.sync_copy(dst2_hbm.at[pl.ds(bchunk, 8)], didx)
        for half in range(2):
            gd = [pltpu.async_copy(y_hbm.at[sidx.at[half * 4 + t]],
                                   rows.at[t], semg) for t in range(4)]
            for d in gd:
                d.wait()
            sd = [pltpu.async_copy(rows.at[t], acc.at[didx.at[half * 4 + t]],
                                   sems, add=True) for t in range(4)]
            for d in sd:
                d.wait()
        return carry

    lax.fori_loop(0, nb, block_step, 0)

    @pl.when((cid == 1) & (sid == _NS - 1))
    def _():
        ntail = _CHUNKS - 781 * 8
        pltpu.sync_copy(src2_hbm.at[pl.ds(781 * 8, ntail)],
                        sidx.at[pl.ds(0, ntail)])
        pltpu.sync_copy(dst2_hbm.at[pl.ds(781 * 8, ntail)],
                        didx.at[pl.ds(0, ntail)])
        gd = [pltpu.async_copy(y_hbm.at[sidx.at[t]], rows.at[t], semg)
              for t in range(ntail)]
        for d in gd:
            d.wait()
        sd = [pltpu.async_copy(rows.at[t], acc.at[didx.at[t]], sems, add=True)
              for t in range(ntail)]
        for d in sd:
            d.wait()

    plsc.subcore_barrier()

    def flush(i, carry):
        r0 = sid * _RPS + i * _ZR
        pltpu.sync_copy(acc.at[pl.ds(r0, _ZR)],
                        out_hbm.at[pl.ds(cid * N_NODES + r0, _ZR)])
        return carry

    lax.fori_loop(0, _NZ, flush, 0)

    @pl.when(sid == 0)
    def _():
        pltpu.sync_copy(acc.at[pl.ds(_REM0, _REM)],
                        out_hbm.at[pl.ds(cid * N_NODES + _REM0, _REM)])


_segsum_call = pl.kernel(
    _segsum_body,
    out_type=jax.ShapeDtypeStruct((_NC * N_NODES, DIM), jnp.float32),
    mesh=_sc_mesh,
    scratch_types=[
        pltpu.VMEM((8, _CH), jnp.int32),
        pltpu.VMEM((8, _CH), jnp.int32),
        pltpu.VMEM((4, _CH, DIM), jnp.float32),
        pltpu.VMEM((_ZR, DIM), jnp.float32),
        pltpu.VMEM_SHARED((N_NODES, DIM), jnp.float32),
        pltpu.SemaphoreType.DMA,
        pltpu.SemaphoreType.DMA,
    ],
    compiler_params=pltpu.CompilerParams(use_tc_tiling_on_sc=False),
)


def _segsum(y, src2, dst2):
    s = _segsum_call(y, src2, dst2)
    return s[:N_NODES], s[N_NODES:]


_BLK = 1000
_NBLK = N_NODES // _BLK


def _matmul_body(x_ref, w_ref, o_ref):
    o_ref[...] = jnp.dot(x_ref[...], w_ref[...],
                         preferred_element_type=jnp.float32)


def _node_matmul(x, w):
    fin = x.shape[1]
    return pl.pallas_call(
        _matmul_body,
        grid=(_NBLK,),
        in_specs=[pl.BlockSpec((_BLK, fin), lambda i: (i, 0)),
                  pl.BlockSpec((fin, DIM), lambda i: (0, 0))],
        out_specs=pl.BlockSpec((_BLK, DIM), lambda i: (i, 0)),
        out_shape=jax.ShapeDtypeStruct((N_NODES, DIM), jnp.float32),
    )(x, w)


def _stats_accum(m, sum_ref, ssq_ref):
    ones = jnp.ones((8, m.shape[0]), jnp.float32)
    ps = jnp.dot(ones, m, preferred_element_type=jnp.float32)
    pq = jnp.dot(ones, m * m, preferred_element_type=jnp.float32)

    @pl.when(pl.program_id(0) == 0)
    def _():
        sum_ref[...] = ps
        ssq_ref[...] = pq

    @pl.when(pl.program_id(0) != 0)
    def _():
        sum_ref[...] += ps
        ssq_ref[...] += pq


def _gin0_body(y_ref, s0_ref, s1_ref, b1_ref, w2_ref, b2_ref,
               m_ref, sum_ref, ssq_ref):
    u = jnp.maximum(y_ref[...] + s0_ref[...] + s1_ref[...] + b1_ref[...], 0.0)
    m = jnp.maximum(
        jnp.dot(u, w2_ref[...], preferred_element_type=jnp.float32)
        + b2_ref[...], 0.0)
    m_ref[...] = m
    _stats_accum(m, sum_ref, ssq_ref)


def _gin_body(h_ref, s0_ref, s1_ref, w1_ref, b1_ref, w2_ref, b2_ref,
              m_ref, sum_ref, ssq_ref):
    t = h_ref[...] + s0_ref[...] + s1_ref[...]
    u = jnp.maximum(
        jnp.dot(t, w1_ref[...], preferred_element_type=jnp.float32)
        + b1_ref[...], 0.0)
    m = jnp.maximum(
        jnp.dot(u, w2_ref[...], preferred_element_type=jnp.float32)
        + b2_ref[...], 0.0)
    m_ref[...] = m
    _stats_accum(m, sum_ref, ssq_ref)


_nodes_spec = pl.BlockSpec((_BLK, DIM), lambda i: (i, 0))
_w_spec = pl.BlockSpec((DIM, DIM), lambda i: (0, 0))
_b_spec = pl.BlockSpec((1, DIM), lambda i: (0, 0))
_stat_spec = pl.BlockSpec((8, DIM), lambda i: (0, 0))
_mlp_out_shapes = (
    jax.ShapeDtypeStruct((N_NODES, DIM), jnp.float32),
    jax.ShapeDtypeStruct((8, DIM), jnp.float32),
    jax.ShapeDtypeStruct((8, DIM), jnp.float32),
)


def _gin_mlp0(y, s0, s1, b1, w2, b2):
    return pl.pallas_call(
        _gin0_body,
        grid=(_NBLK,),
        in_specs=[_nodes_spec, _nodes_spec, _nodes_spec,
                  _b_spec, _w_spec, _b_spec],
        out_specs=(_nodes_spec, _stat_spec, _stat_spec),
        out_shape=_mlp_out_shapes,
    )(y, s0, s1, b1, w2, b2)


def _gin_mlp(h, s0, s1, w1, b1, w2, b2):
    return pl.pallas_call(
        _gin_body,
        grid=(_NBLK,),
        in_specs=[_nodes_spec, _nodes_spec, _nodes_spec,
                  _w_spec, _b_spec, _w_spec, _b_spec],
        out_specs=(_nodes_spec, _stat_spec, _stat_spec),
        out_shape=_mlp_out_shapes,
    )(h, s0, s1, w1, b1, w2, b2)


def _bn_scale(sum_v, ssq_v, g_v, b_v):
    s = jnp.sum(sum_v, axis=0, keepdims=True)
    q = jnp.sum(ssq_v, axis=0, keepdims=True)
    mu = s * (1.0 / N_NODES)
    var = q * (1.0 / N_NODES) - mu * mu
    a = g_v * lax.rsqrt(var + 1e-5)
    c = b_v - mu * a
    return a, c


def _bn_body(m_ref, sum_ref, ssq_ref, g_ref, b_ref, h_ref):
    a, c = _bn_scale(sum_ref[...], ssq_ref[...], g_ref[...], b_ref[...])
    h_ref[...] = m_ref[...] * a + c


def _bn_apply(m, s8, q8, g, b):
    return pl.pallas_call(
        _bn_body,
        grid=(_NBLK,),
        in_specs=[_nodes_spec, _stat_spec, _stat_spec, _b_spec, _b_spec],
        out_specs=_nodes_spec,
        out_shape=jax.ShapeDtypeStruct((N_NODES, DIM), jnp.float32),
    )(m, s8, q8, g, b)


def _bn_pool_body(m_ref, sum_ref, ssq_ref, g_ref, b_ref, bt_ref, p_ref):
    a, c = _bn_scale(sum_ref[...], ssq_ref[...], g_ref[...], b_ref[...])
    h = m_ref[...] * a + c
    brow = bt_ref[...].reshape(1, _BLK)
    oh = (lax.broadcasted_iota(jnp.int32, (N_GRAPHS, _BLK), 0)
          == brow).astype(jnp.float32)
    pp = jnp.dot(oh, h, preferred_element_type=jnp.float32)

    @pl.when(pl.program_id(0) == 0)
    def _():
        p_ref[...] = pp

    @pl.when(pl.program_id(0) != 0)
    def _():
        p_ref[...] += pp


def _bn_pool(m, s8, q8, g, b, batch3):
    return pl.pallas_call(
        _bn_pool_body,
        grid=(_NBLK,),
        in_specs=[_nodes_spec, _stat_spec, _stat_spec, _b_spec, _b_spec,
                  pl.BlockSpec((1, 1, _BLK), lambda i: (i, 0, 0))],
        out_specs=pl.BlockSpec((N_GRAPHS, DIM), lambda i: (0, 0)),
        out_shape=jax.ShapeDtypeStruct((N_GRAPHS, DIM), jnp.float32),
    )(m, s8, q8, g, b, batch3)


def _heads_body(t2_ref, t1_ref, w2s_ref, a2t_ref, b2c_ref, w1s_ref, b1c_ref,
                c2_ref, c1_ref):
    t2 = t2_ref[...].reshape(1, 1000)
    oh = (lax.broadcasted_iota(jnp.int32, (32, 1000), 0)
          == t2).astype(jnp.float32)
    c2 = jnp.zeros((32, 121), jnp.float32)
    for k in range(8):
        bk = lax.dot_general(w2s_ref[k], oh, (((1,), (1,)), ((), ())),
                             preferred_element_type=jnp.float32)
        c2 = c2 + jnp.dot(bk, a2t_ref[k], preferred_element_type=jnp.float32)
    c2_ref[...] = (c2 + b2c_ref[...]).reshape(1, 32, 121)

    t1 = t1_ref[...].reshape(20, 24)
    c1 = jnp.zeros((32, 17), jnp.float32)
    for k in range(8):
        c1 = c1 + jnp.dot(w1s_ref[k], t1[:, k:k + 17],
                          preferred_element_type=jnp.float32)
    c1_ref[...] = (c1 + b1c_ref[...]).reshape(1, 32, 17)


def _conv_heads(t2r, t1, w2s, a2t, b2c, w1s, b1c):
    return pl.pallas_call(
        _heads_body,
        grid=(N_GRAPHS,),
        in_specs=[
            pl.BlockSpec((1, 1, 1000), lambda i: (i, 0, 0)),
            pl.BlockSpec((1, 20, 24), lambda i: (i, 0, 0)),
            pl.BlockSpec((8, 32, 1000), lambda i: (0, 0, 0)),
            pl.BlockSpec((8, 32, 121), lambda i: (0, 0, 0)),
            pl.BlockSpec((32, 1), lambda i: (0, 0)),
            pl.BlockSpec((8, 32, 20), lambda i: (0, 0, 0)),
            pl.BlockSpec((32, 1), lambda i: (0, 0)),
        ],
        out_specs=(pl.BlockSpec((1, 32, 121), lambda i: (i, 0, 0)),
                   pl.BlockSpec((1, 32, 17), lambda i: (i, 0, 0))),
        out_shape=(jax.ShapeDtypeStruct((N_GRAPHS, 32, 121), jnp.float32),
                   jax.ShapeDtypeStruct((N_GRAPHS, 32, 17), jnp.float32)),
    )(t2r, t1, w2s, a2t, b2c, w1s, b1c)


def _final_body(p_ref, c1f_ref, c2f_ref, wxd_ref, bxd_ref, w1x_ref, b1x_ref,
                w2x_ref, b2x_ref, wa_ref, wb_ref, wc_ref, bf1_ref,
                wf2_ref, bf2_ref, wo_ref, bo_ref, o_ref):
    f32 = jnp.float32
    xd = jnp.maximum(
        jnp.dot(p_ref[...], wxd_ref[...], preferred_element_type=f32)
        + bxd_ref[...], 0.0)
    xt1 = jnp.dot(c1f_ref[...], w1x_ref[...],
                  preferred_element_type=f32) + b1x_ref[...]
    xt2 = jnp.dot(c2f_ref[...], w2x_ref[...],
                  preferred_element_type=f32) + b2x_ref[...]
    z = jnp.maximum(
        jnp.dot(xd, wa_ref[...], preferred_element_type=f32)
        + jnp.dot(xt1, wb_ref[...], preferred_element_type=f32)
        + jnp.dot(xt2, wc_ref[...], preferred_element_type=f32)
        + bf1_ref[...], 0.0)
    z2 = jnp.maximum(
        jnp.dot(z, wf2_ref[...], preferred_element_type=f32)
        + bf2_ref[...], 0.0)
    o_ref[...] = jnp.dot(z2, wo_ref[...],
                         preferred_element_type=f32) + bo_ref[...]


def _final_mlp(pooled, c1f, c2f, wxd, bxd, w1x, b1x, w2x, b2x,
               wa, wb, wc, bf1, wf2, bf2, wo, bo):
    return pl.pallas_call(
        _final_body,
        out_shape=jax.ShapeDtypeStruct((N_GRAPHS, 1), jnp.float32),
    )(pooled, c1f, c2f, wxd, bxd, w1x, b1x, w2x, b2x,
      wa, wb, wc, bf1, wf2, bf2, wo, bo)


def kernel(x, edge_index, batch, target1, target2, params):
    p = params
    src2 = edge_index[0].reshape(_CHUNKS, _CH)
    dst2 = edge_index[1].reshape(_CHUNKS, _CH)

    def row(v):
        return v.reshape(1, DIM)

    h = None
    pooled = None
    batch3 = batch.astype(jnp.int32).reshape(_NBLK, 1, _BLK)
    for i in range(5):
        if i == 0:
            y = _node_matmul(x, p['gin0_w1'])
            s0, s1 = _segsum(y, src2, dst2)
            m, s8, q8 = _gin_mlp0(y, s0, s1, row(p['gin0_b1']),
                                  p['gin0_w2'], row(p['gin0_b2']))
        else:
            s0, s1 = _segsum(h, src2, dst2)
            m, s8, q8 = _gin_mlp(h, s0, s1, p['gin%d_w1' % i],
                                 row(p['gin%d_b1' % i]), p['gin%d_w2' % i],
                                 row(p['gin%d_b2' % i]))
        if i < 4:
            h = _bn_apply(m, s8, q8, row(p['bn%d_g' % i]), row(p['bn%d_b' % i]))
        else:
            pooled = _bn_pool(m, s8, q8, row(p['bn%d_g' % i]),
                              row(p['bn%d_b' % i]), batch3)

    t2r = target2.astype(jnp.int32).reshape(N_GRAPHS, 1, 1000)
    epad = jnp.concatenate(
        [p['emb'], jnp.zeros((6, 128), jnp.float32)], axis=0)
    a2t = jnp.stack([epad[:, k:k + 121] for k in range(8)])
    w2s = jnp.transpose(p['cxt2_w'], (2, 0, 1))
    w1s = jnp.transpose(p['cxt1_w'], (2, 0, 1))
    c2, c1 = _conv_heads(t2r, target1, w2s, a2t,
                         p['cxt2_b'].reshape(32, 1), w1s,
                         p['cxt1_b'].reshape(32, 1))
    c2f = c2.reshape(N_GRAPHS, 32 * 121)
    c1f = c1.reshape(N_GRAPHS, 32 * 17)

    fw = p['fc1_w']
    out = _final_mlp(
        pooled, c1f, c2f,
        p['fc1_xd_w'], p['fc1_xd_b'].reshape(1, -1),
        p['fc1_xt_w'], p['fc1_xt_b'].reshape(1, -1),
        p['fc2_xt_w'], p['fc2_xt_b'].reshape(1, -1),
        fw[0:128], fw[128:256], fw[256:384], p['fc1_b'].reshape(1, -1),
        p['fc2_w'], p['fc2_b'].reshape(1, -1),
        p['out_w'], p['out_b'].reshape(1, -1))
    return out


# speedup vs baseline: 11.4778x; 1.2089x over previous
import jax
import jax.numpy as jnp
from jax import lax
from jax.experimental import pallas as pl
from jax.experimental.pallas import tpu as pltpu
from jax.experimental.pallas import tpu_sc as plsc

N_GRAPHS = 128
N_NODES = 50000
DIM = 32
N_EDGES = 800000

_NC, _NS = 2, 16
_CH = 128
_CHUNKS = N_EDGES // _CH
_CPC = _CHUNKS // _NC
_RPS = 3120
_ZR = 120
_NZ = _RPS // _ZR
_REM0 = _NS * _RPS
_REM = N_NODES - _REM0

_sc_mesh = plsc.VectorSubcoreMesh(
    core_axis_name="c", subcore_axis_name="s", num_cores=_NC, num_subcores=_NS)


def _segsum_body(y_hbm, src2_hbm, dst2_hbm, out_hbm,
                 sidx, didx, rows, zbuf, acc, semg, sems):
    cid = lax.axis_index("c")
    sid = lax.axis_index("s")

    zv = jnp.zeros((16,), jnp.float32)

    def zb(i, carry):
        zbuf[i, pl.ds(0, 16)] = zv
        zbuf[i, pl.ds(16, 16)] = zv
        return carry

    lax.fori_loop(0, _ZR, zb, 0)

    def zacc(i, carry):
        pltpu.sync_copy(zbuf, acc.at[pl.ds(sid * _RPS + i * _ZR, _ZR)])
        return carry

    lax.fori_loop(0, _NZ, zacc, 0)

    @pl.when(sid == 0)
    def _():
        pltpu.sync_copy(zbuf.at[pl.ds(0, _REM)], acc.at[pl.ds(_REM0, _REM)])

    plsc.subcore_barrier()

    extra = jnp.where(cid == 0, 7, 6)
    nb = 24 + jnp.where(sid < extra, 1, 0)
    core_base = cid * 3128

    def block_step(k, carry):
        bchunk = core_base + (sid + _NS * k) * 8
        pltpu.sync_copy(src2_hbm.at[pl.ds(bchunk, 8)], sidx)
        pltpu.sync_copy(dst2_hbm.at[pl.ds(bchunk, 8)], didx)
        for half in range(2):
            gd = [pltpu.async_copy(y_hbm.at[sidx.at[half * 4 + t]],
                                   rows.at[t], semg) for t in range(4)]
            for d in gd:
                d.wait()
            sd = [pltpu.async_copy(rows.at[t], acc.at[didx.at[half * 4 + t]],
                                   sems, add=True) for t in range(4)]
            for d in sd:
                d.wait()
        return carry

    lax.fori_loop(0, nb, block_step, 0)

    @pl.when((cid == 1) & (sid == _NS - 1))
    def _():
        ntail = _CHUNKS - 781 * 8
        pltpu.sync_copy(src2_hbm.at[pl.ds(781 * 8, ntail)],
                        sidx.at[pl.ds(0, ntail)])
        pltpu.sync_copy(dst2_hbm.at[pl.ds(781 * 8, ntail)],
                        didx.at[pl.ds(0, ntail)])
        gd = [pltpu.async_copy(y_hbm.at[sidx.at[t]], rows.at[t], semg)
              for t in range(ntail)]
        for d in gd:
            d.wait()
        sd = [pltpu.async_copy(rows.at[t], acc.at[didx.at[t]], sems, add=True)
              for t in range(ntail)]
        for d in sd:
            d.wait()

    plsc.subcore_barrier()

    def flush(i, carry):
        r0 = sid * _RPS + i * _ZR
        pltpu.sync_copy(acc.at[pl.ds(r0, _ZR)],
                        out_hbm.at[pl.ds(cid * N_NODES + r0, _ZR)])
        return carry

    lax.fori_loop(0, _NZ, flush, 0)

    @pl.when(sid == 0)
    def _():
        pltpu.sync_copy(acc.at[pl.ds(_REM0, _REM)],
                        out_hbm.at[pl.ds(cid * N_NODES + _REM0, _REM)])


_segsum_call = pl.kernel(
    _segsum_body,
    out_type=jax.ShapeDtypeStruct((_NC * N_NODES, DIM), jnp.float32),
    mesh=_sc_mesh,
    scratch_types=[
        pltpu.VMEM((8, _CH), jnp.int32),
        pltpu.VMEM((8, _CH), jnp.int32),
        pltpu.VMEM((4, _CH, DIM), jnp.float32),
        pltpu.VMEM((_ZR, DIM), jnp.float32),
        pltpu.VMEM_SHARED((N_NODES, DIM), jnp.float32),
        pltpu.SemaphoreType.DMA,
        pltpu.SemaphoreType.DMA,
    ],
    compiler_params=pltpu.CompilerParams(use_tc_tiling_on_sc=False),
)


_BLK = 2000
_NBLK = N_NODES // _BLK


def _matmul_body(x_ref, w_ref, o_ref):
    o_ref[...] = jnp.dot(x_ref[...], w_ref[...],
                         preferred_element_type=jnp.float32)


def _node_matmul(x, w):
    fin = x.shape[1]
    return pl.pallas_call(
        _matmul_body,
        grid=(_NBLK,),
        in_specs=[pl.BlockSpec((_BLK, fin), lambda i: (i, 0)),
                  pl.BlockSpec((fin, DIM), lambda i: (0, 0))],
        out_specs=pl.BlockSpec((_BLK, DIM), lambda i: (i, 0)),
        out_shape=jax.ShapeDtypeStruct((N_NODES, DIM), jnp.float32),
    )(x, w)


def _stats_accum(m, sum_ref, ssq_ref):
    ones = jnp.ones((8, m.shape[0]), jnp.float32)
    ps = jnp.dot(ones, m, preferred_element_type=jnp.float32)
    pq = jnp.dot(ones, m * m, preferred_element_type=jnp.float32)

    @pl.when(pl.program_id(0) == 0)
    def _():
        sum_ref[...] = ps
        ssq_ref[...] = pq

    @pl.when(pl.program_id(0) != 0)
    def _():
        sum_ref[...] += ps
        ssq_ref[...] += pq


def _gin0_body(y_ref, s0_ref, s1_ref, b1_ref, w2_ref, b2_ref,
               m_ref, sum_ref, ssq_ref):
    u = jnp.maximum(y_ref[...] + s0_ref[...] + s1_ref[...] + b1_ref[...], 0.0)
    m = jnp.maximum(
        jnp.dot(u, w2_ref[...], preferred_element_type=jnp.float32)
        + b2_ref[...], 0.0)
    m_ref[...] = m
    _stats_accum(m, sum_ref, ssq_ref)


def _gin_body(m_ref, s0_ref, s1_ref, d0_ref, d1_ref, sum_ref, ssq_ref,
              g_ref, b_ref, w1_ref, b1_ref, w2_ref, b2_ref,
              mo_ref, sumo_ref, ssqo_ref):
    a, c = _bn_scale(sum_ref[...], ssq_ref[...], g_ref[...], b_ref[...])
    t = (a * (m_ref[...] + s0_ref[...] + s1_ref[...])
         + c * (1.0 + d0_ref[...] + d1_ref[...]))
    u = jnp.maximum(
        jnp.dot(t, w1_ref[...], preferred_element_type=jnp.float32)
        + b1_ref[...], 0.0)
    m = jnp.maximum(
        jnp.dot(u, w2_ref[...], preferred_element_type=jnp.float32)
        + b2_ref[...], 0.0)
    mo_ref[...] = m
    _stats_accum(m, sumo_ref, ssqo_ref)


_nodes_spec = pl.BlockSpec((_BLK, DIM), lambda i: (i, 0))
_part0_spec = pl.BlockSpec((_BLK, DIM), lambda i: (i, 0))
_part1_spec = pl.BlockSpec((_BLK, DIM), lambda i: (i + _NBLK, 0))
_w_spec = pl.BlockSpec((DIM, DIM), lambda i: (0, 0))
_b_spec = pl.BlockSpec((1, DIM), lambda i: (0, 0))
_stat_spec = pl.BlockSpec((8, DIM), lambda i: (0, 0))
_mlp_out_shapes = (
    jax.ShapeDtypeStruct((N_NODES, DIM), jnp.float32),
    jax.ShapeDtypeStruct((8, DIM), jnp.float32),
    jax.ShapeDtypeStruct((8, DIM), jnp.float32),
)


def _gin_mlp0(y, s, b1, w2, b2):
    return pl.pallas_call(
        _gin0_body,
        grid=(_NBLK,),
        in_specs=[_nodes_spec, _part0_spec, _part1_spec,
                  _b_spec, _w_spec, _b_spec],
        out_specs=(_nodes_spec, _stat_spec, _stat_spec),
        out_shape=_mlp_out_shapes,
    )(y, s, s, b1, w2, b2)


def _gin_mlp(m, s, d, s8, q8, g, b, w1, b1, w2, b2):
    return pl.pallas_call(
        _gin_body,
        grid=(_NBLK,),
        in_specs=[_nodes_spec, _part0_spec, _part1_spec,
                  _part0_spec, _part1_spec,
                  _stat_spec, _stat_spec, _b_spec, _b_spec,
                  _w_spec, _b_spec, _w_spec, _b_spec],
        out_specs=(_nodes_spec, _stat_spec, _stat_spec),
        out_shape=_mlp_out_shapes,
    )(m, s, s, d, d, s8, q8, g, b, w1, b1, w2, b2)


def _bn_scale(sum_v, ssq_v, g_v, b_v):
    s = jnp.sum(sum_v, axis=0, keepdims=True)
    q = jnp.sum(ssq_v, axis=0, keepdims=True)
    mu = s * (1.0 / N_NODES)
    var = q * (1.0 / N_NODES) - mu * mu
    a = g_v * lax.rsqrt(var + 1e-5)
    c = b_v - mu * a
    return a, c


def _bn_pool_body(m_ref, sum_ref, ssq_ref, g_ref, b_ref, bt_ref, p_ref):
    a, c = _bn_scale(sum_ref[...], ssq_ref[...], g_ref[...], b_ref[...])
    h = m_ref[...] * a + c
    brow = bt_ref[...].reshape(1, _BLK)
    oh = (lax.broadcasted_iota(jnp.int32, (N_GRAPHS, _BLK), 0)
          == brow).astype(jnp.float32)
    pp = jnp.dot(oh, h, preferred_element_type=jnp.float32)

    @pl.when(pl.program_id(0) == 0)
    def _():
        p_ref[...] = pp

    @pl.when(pl.program_id(0) != 0)
    def _():
        p_ref[...] += pp


def _bn_pool(m, s8, q8, g, b, batch3):
    return pl.pallas_call(
        _bn_pool_body,
        grid=(_NBLK,),
        in_specs=[_nodes_spec, _stat_spec, _stat_spec, _b_spec, _b_spec,
                  pl.BlockSpec((1, 1, _BLK), lambda i: (i, 0, 0))],
        out_specs=pl.BlockSpec((N_GRAPHS, DIM), lambda i: (0, 0)),
        out_shape=jax.ShapeDtypeStruct((N_GRAPHS, DIM), jnp.float32),
    )(m, s8, q8, g, b, batch3)


_HG = 8


def _heads_body(t2_ref, t1_ref, w2s_ref, a2t_ref, b2c_ref, w1s_ref, b1c_ref,
                c2_ref, c1_ref):
    t2 = t2_ref[...].reshape(_HG, 1000)
    t1 = t1_ref[...]
    for g in range(_HG):
        t2g = t2[g:g + 1, :]
        oh = (lax.broadcasted_iota(jnp.int32, (32, 1000), 0)
              == t2g).astype(jnp.float32)
        c2 = jnp.zeros((32, 121), jnp.float32)
        for k in range(8):
            bk = lax.dot_general(w2s_ref[k], oh, (((1,), (1,)), ((), ())),
                                 preferred_element_type=jnp.float32)
            c2 = c2 + jnp.dot(bk, a2t_ref[k],
                              preferred_element_type=jnp.float32)
        c2_ref[g] = c2 + b2c_ref[...]

        t1g = t1[g]
        c1 = jnp.zeros((32, 17), jnp.float32)
        for k in range(8):
            c1 = c1 + jnp.dot(w1s_ref[k], t1g[:, k:k + 17],
                              preferred_element_type=jnp.float32)
        c1_ref[g] = c1 + b1c_ref[...]


def _conv_heads(t2r, t1, w2s, a2t, b2c, w1s, b1c):
    return pl.pallas_call(
        _heads_body,
        grid=(N_GRAPHS // _HG,),
        in_specs=[
            pl.BlockSpec((_HG, 1, 1000), lambda i: (i, 0, 0)),
            pl.BlockSpec((_HG, 20, 24), lambda i: (i, 0, 0)),
            pl.BlockSpec((8, 32, 1000), lambda i: (0, 0, 0)),
            pl.BlockSpec((8, 32, 121), lambda i: (0, 0, 0)),
            pl.BlockSpec((32, 1), lambda i: (0, 0)),
            pl.BlockSpec((8, 32, 20), lambda i: (0, 0, 0)),
            pl.BlockSpec((32, 1), lambda i: (0, 0)),
        ],
        out_specs=(pl.BlockSpec((_HG, 32, 121), lambda i: (i, 0, 0)),
                   pl.BlockSpec((_HG, 32, 17), lambda i: (i, 0, 0))),
        out_shape=(jax.ShapeDtypeStruct((N_GRAPHS, 32, 121), jnp.float32),
                   jax.ShapeDtypeStruct((N_GRAPHS, 32, 17), jnp.float32)),
    )(t2r, t1, w2s, a2t, b2c, w1s, b1c)


def _final_body(p_ref, c1f_ref, c2f_ref, wxd_ref, bxd_ref, w1x_ref, b1x_ref,
                w2x_ref, b2x_ref, wa_ref, wb_ref, wc_ref, bf1_ref,
                wf2_ref, bf2_ref, wo_ref, bo_ref, o_ref):
    f32 = jnp.float32
    xd = jnp.maximum(
        jnp.dot(p_ref[...], wxd_ref[...], preferred_element_type=f32)
        + bxd_ref[...], 0.0)
    xt1 = jnp.dot(c1f_ref[...], w1x_ref[...],
                  preferred_element_type=f32) + b1x_ref[...]
    xt2 = jnp.dot(c2f_ref[...], w2x_ref[...],
                  preferred_element_type=f32) + b2x_ref[...]
    z = jnp.maximum(
        jnp.dot(xd, wa_ref[...], preferred_element_type=f32)
        + jnp.dot(xt1, wb_ref[...], preferred_element_type=f32)
        + jnp.dot(xt2, wc_ref[...], preferred_element_type=f32)
        + bf1_ref[...], 0.0)
    z2 = jnp.maximum(
        jnp.dot(z, wf2_ref[...], preferred_element_type=f32)
        + bf2_ref[...], 0.0)
    o_ref[...] = jnp.dot(z2, wo_ref[...],
                         preferred_element_type=f32) + bo_ref[...]


def _final_mlp(pooled, c1f, c2f, wxd, bxd, w1x, b1x, w2x, b2x,
               wa, wb, wc, bf1, wf2, bf2, wo, bo):
    return pl.pallas_call(
        _final_body,
        out_shape=jax.ShapeDtypeStruct((N_GRAPHS, 1), jnp.float32),
    )(pooled, c1f, c2f, wxd, bxd, w1x, b1x, w2x, b2x,
      wa, wb, wc, bf1, wf2, bf2, wo, bo)


def kernel(x, edge_index, batch, target1, target2, params):
    p = params
    src2 = edge_index[0].reshape(_CHUNKS, _CH)
    dst2 = edge_index[1].reshape(_CHUNKS, _CH)

    def row(v):
        return v.reshape(1, DIM)

    batch3 = batch.astype(jnp.int32).reshape(_NBLK, 1, _BLK)
    deg = _segsum_call(jnp.ones((N_NODES, DIM), jnp.float32), src2, dst2)
    y = _node_matmul(x, p['gin0_w1'])
    s = _segsum_call(y, src2, dst2)
    m, s8, q8 = _gin_mlp0(y, s, row(p['gin0_b1']),
                          p['gin0_w2'], row(p['gin0_b2']))
    for i in range(1, 5):
        s = _segsum_call(m, src2, dst2)
        m, s8, q8 = _gin_mlp(m, s, deg, s8, q8,
                             row(p['bn%d_g' % (i - 1)]),
                             row(p['bn%d_b' % (i - 1)]),
                             p['gin%d_w1' % i], row(p['gin%d_b1' % i]),
                             p['gin%d_w2' % i], row(p['gin%d_b2' % i]))
    pooled = _bn_pool(m, s8, q8, row(p['bn4_g']), row(p['bn4_b']), batch3)

    t2r = target2.astype(jnp.int32).reshape(N_GRAPHS, 1, 1000)
    epad = jnp.concatenate(
        [p['emb'], jnp.zeros((6, 128), jnp.float32)], axis=0)
    a2t = jnp.stack([epad[:, k:k + 121] for k in range(8)])
    w2s = jnp.transpose(p['cxt2_w'], (2, 0, 1))
    w1s = jnp.transpose(p['cxt1_w'], (2, 0, 1))
    c2, c1 = _conv_heads(t2r, target1, w2s, a2t,
                         p['cxt2_b'].reshape(32, 1), w1s,
                         p['cxt1_b'].reshape(32, 1))
    c2f = c2.reshape(N_GRAPHS, 32 * 121)
    c1f = c1.reshape(N_GRAPHS, 32 * 17)

    fw = p['fc1_w']
    out = _final_mlp(
        pooled, c1f, c2f,
        p['fc1_xd_w'], p['fc1_xd_b'].reshape(1, -1),
        p['fc1_xt_w'], p['fc1_xt_b'].reshape(1, -1),
        p['fc2_xt_w'], p['fc2_xt_b'].reshape(1, -1),
        fw[0:128], fw[128:256], fw[256:384], p['fc1_b'].reshape(1, -1),
        p['fc2_w'], p['fc2_b'].reshape(1, -1),
        p['out_w'], p['out_b'].reshape(1, -1))
    return out


# speedup vs baseline: 11.6719x; 1.0169x over previous
import jax
import jax.numpy as jnp
from jax import lax
from jax.experimental import pallas as pl
from jax.experimental.pallas import tpu as pltpu
from jax.experimental.pallas import tpu_sc as plsc

N_GRAPHS = 128
N_NODES = 50000
DIM = 32
N_EDGES = 800000

_NC, _NS = 2, 16
_CH = 128
_CHUNKS = N_EDGES // _CH
_CPC = _CHUNKS // _NC
_RPS = 3120
_ZR = 120
_NZ = _RPS // _ZR
_REM0 = _NS * _RPS
_REM = N_NODES - _REM0

_sc_mesh = plsc.VectorSubcoreMesh(
    core_axis_name="c", subcore_axis_name="s", num_cores=_NC, num_subcores=_NS)


def _segsum_body(y_hbm, src2_hbm, dst2_hbm, out_hbm,
                 sidx, didx, rowsa, rowsb, zbuf, acc,
                 semga, semgb, semsa, semsb):
    cid = lax.axis_index("c")
    sid = lax.axis_index("s")

    zv = jnp.zeros((16,), jnp.float32)

    def zb(i, carry):
        zbuf[i, pl.ds(0, 16)] = zv
        zbuf[i, pl.ds(16, 16)] = zv
        return carry

    lax.fori_loop(0, _ZR, zb, 0)

    def zacc(i, carry):
        pltpu.sync_copy(zbuf, acc.at[pl.ds(sid * _RPS + i * _ZR, _ZR)])
        return carry

    lax.fori_loop(0, _NZ, zacc, 0)

    @pl.when(sid == 0)
    def _():
        pltpu.sync_copy(zbuf.at[pl.ds(0, _REM)], acc.at[pl.ds(_REM0, _REM)])

    plsc.subcore_barrier()

    extra = jnp.where(cid == 0, 7, 6)
    nb = 24 + jnp.where(sid < extra, 1, 0)
    core_base = cid * 3128

    bufs = (rowsa, rowsb)
    gsems = (semga, semgb)
    ssems = (semsa, semsb)

    def _fire_g(pr):
        b = pr % 2
        return [pltpu.async_copy(y_hbm.at[sidx.at[2 * pr + t]],
                                 bufs[b].at[t], gsems[b]) for t in range(2)]

    def _fire_s(pr):
        b = pr % 2
        return [pltpu.async_copy(bufs[b].at[t],
                                 acc.at[didx.at[2 * pr + t]],
                                 ssems[b], add=True) for t in range(2)]

    def block_step(k, carry):
        bchunk = core_base + (sid + _NS * k) * 8
        pltpu.sync_copy(src2_hbm.at[pl.ds(bchunk, 8)], sidx)
        pltpu.sync_copy(dst2_hbm.at[pl.ds(bchunk, 8)], didx)
        g0 = _fire_g(0)
        g1 = _fire_g(1)
        for d in g0:
            d.wait()
        s0 = _fire_s(0)
        for d in g1:
            d.wait()
        s1 = _fire_s(1)
        for d in s0:
            d.wait()
        g2 = _fire_g(2)
        for d in s1:
            d.wait()
        g3 = _fire_g(3)
        for d in g2:
            d.wait()
        s2 = _fire_s(2)
        for d in g3:
            d.wait()
        s3 = _fire_s(3)
        for d in s2:
            d.wait()
        for d in s3:
            d.wait()
        return carry

    lax.fori_loop(0, nb, block_step, 0)

    @pl.when((cid == 1) & (sid == _NS - 1))
    def _():
        ntail = _CHUNKS - 781 * 8
        pltpu.sync_copy(src2_hbm.at[pl.ds(781 * 8, ntail)],
                        sidx.at[pl.ds(0, ntail)])
        pltpu.sync_copy(dst2_hbm.at[pl.ds(781 * 8, ntail)],
                        didx.at[pl.ds(0, ntail)])
        gd = [pltpu.async_copy(y_hbm.at[sidx.at[t]], rowsa.at[t], semga)
              for t in range(ntail)]
        for d in gd:
            d.wait()
        sd = [pltpu.async_copy(rowsa.at[t], acc.at[didx.at[t]], semsa,
                               add=True) for t in range(ntail)]
        for d in sd:
            d.wait()

    plsc.subcore_barrier()

    def flush(i, carry):
        r0 = sid * _RPS + i * _ZR
        pltpu.sync_copy(acc.at[pl.ds(r0, _ZR)],
                        out_hbm.at[pl.ds(cid * N_NODES + r0, _ZR)])
        return carry

    lax.fori_loop(0, _NZ, flush, 0)

    @pl.when(sid == 0)
    def _():
        pltpu.sync_copy(acc.at[pl.ds(_REM0, _REM)],
                        out_hbm.at[pl.ds(cid * N_NODES + _REM0, _REM)])


_segsum_call = pl.kernel(
    _segsum_body,
    out_type=jax.ShapeDtypeStruct((_NC * N_NODES, DIM), jnp.float32),
    mesh=_sc_mesh,
    scratch_types=[
        pltpu.VMEM((8, _CH), jnp.int32),
        pltpu.VMEM((8, _CH), jnp.int32),
        pltpu.VMEM((2, _CH, DIM), jnp.float32),
        pltpu.VMEM((2, _CH, DIM), jnp.float32),
        pltpu.VMEM((_ZR, DIM), jnp.float32),
        pltpu.VMEM_SHARED((N_NODES, DIM), jnp.float32),
        pltpu.SemaphoreType.DMA,
        pltpu.SemaphoreType.DMA,
        pltpu.SemaphoreType.DMA,
        pltpu.SemaphoreType.DMA,
    ],
    compiler_params=pltpu.CompilerParams(use_tc_tiling_on_sc=False),
)


_BLK = 2000
_NBLK = N_NODES // _BLK


def _matmul_body(x_ref, w_ref, o_ref):
    o_ref[...] = jnp.dot(x_ref[...], w_ref[...],
                         preferred_element_type=jnp.float32)


def _node_matmul(x, w):
    fin = x.shape[1]
    return pl.pallas_call(
        _matmul_body,
        grid=(_NBLK,),
        in_specs=[pl.BlockSpec((_BLK, fin), lambda i: (i, 0)),
                  pl.BlockSpec((fin, DIM), lambda i: (0, 0))],
        out_specs=pl.BlockSpec((_BLK, DIM), lambda i: (i, 0)),
        out_shape=jax.ShapeDtypeStruct((N_NODES, DIM), jnp.float32),
    )(x, w)


def _stats_accum(m, sum_ref, ssq_ref):
    ones = jnp.ones((8, m.shape[0]), jnp.float32)
    ps = jnp.dot(ones, m, preferred_element_type=jnp.float32)
    pq = jnp.dot(ones, m * m, preferred_element_type=jnp.float32)

    @pl.when(pl.program_id(0) == 0)
    def _():
        sum_ref[...] = ps
        ssq_ref[...] = pq

    @pl.when(pl.program_id(0) != 0)
    def _():
        sum_ref[...] += ps
        ssq_ref[...] += pq


def _gin0_body(y_ref, s0_ref, s1_ref, b1_ref, w2_ref, b2_ref,
               m_ref, sum_ref, ssq_ref):
    u = jnp.maximum(y_ref[...] + s0_ref[...] + s1_ref[...] + b1_ref[...], 0.0)
    m = jnp.maximum(
        jnp.dot(u, w2_ref[...], preferred_element_type=jnp.float32)
        + b2_ref[...], 0.0)
    m_ref[...] = m
    _stats_accum(m, sum_ref, ssq_ref)


def _gin_body(m_ref, s0_ref, s1_ref, d0_ref, d1_ref, sum_ref, ssq_ref,
              g_ref, b_ref, w1_ref, b1_ref, w2_ref, b2_ref,
              mo_ref, sumo_ref, ssqo_ref):
    a, c = _bn_scale(sum_ref[...], ssq_ref[...], g_ref[...], b_ref[...])
    t = (a * (m_ref[...] + s0_ref[...] + s1_ref[...])
         + c * (1.0 + d0_ref[...] + d1_ref[...]))
    u = jnp.maximum(
        jnp.dot(t, w1_ref[...], preferred_element_type=jnp.float32)
        + b1_ref[...], 0.0)
    m = jnp.maximum(
        jnp.dot(u, w2_ref[...], preferred_element_type=jnp.float32)
        + b2_ref[...], 0.0)
    mo_ref[...] = m
    _stats_accum(m, sumo_ref, ssqo_ref)


_nodes_spec = pl.BlockSpec((_BLK, DIM), lambda i: (i, 0))
_part0_spec = pl.BlockSpec((_BLK, DIM), lambda i: (i, 0))
_part1_spec = pl.BlockSpec((_BLK, DIM), lambda i: (i + _NBLK, 0))
_w_spec = pl.BlockSpec((DIM, DIM), lambda i: (0, 0))
_b_spec = pl.BlockSpec((1, DIM), lambda i: (0, 0))
_stat_spec = pl.BlockSpec((8, DIM), lambda i: (0, 0))
_mlp_out_shapes = (
    jax.ShapeDtypeStruct((N_NODES, DIM), jnp.float32),
    jax.ShapeDtypeStruct((8, DIM), jnp.float32),
    jax.ShapeDtypeStruct((8, DIM), jnp.float32),
)


def _gin_mlp0(y, s, b1, w2, b2):
    return pl.pallas_call(
        _gin0_body,
        grid=(_NBLK,),
        in_specs=[_nodes_spec, _part0_spec, _part1_spec,
                  _b_spec, _w_spec, _b_spec],
        out_specs=(_nodes_spec, _stat_spec, _stat_spec),
        out_shape=_mlp_out_shapes,
    )(y, s, s, b1, w2, b2)


def _gin_mlp(m, s, d, s8, q8, g, b, w1, b1, w2, b2):
    return pl.pallas_call(
        _gin_body,
        grid=(_NBLK,),
        in_specs=[_nodes_spec, _part0_spec, _part1_spec,
                  _part0_spec, _part1_spec,
                  _stat_spec, _stat_spec, _b_spec, _b_spec,
                  _w_spec, _b_spec, _w_spec, _b_spec],
        out_specs=(_nodes_spec, _stat_spec, _stat_spec),
        out_shape=_mlp_out_shapes,
    )(m, s, s, d, d, s8, q8, g, b, w1, b1, w2, b2)


def _bn_scale(sum_v, ssq_v, g_v, b_v):
    s = jnp.sum(sum_v, axis=0, keepdims=True)
    q = jnp.sum(ssq_v, axis=0, keepdims=True)
    mu = s * (1.0 / N_NODES)
    var = q * (1.0 / N_NODES) - mu * mu
    a = g_v * lax.rsqrt(var + 1e-5)
    c = b_v - mu * a
    return a, c


def _bn_pool_body(m_ref, sum_ref, ssq_ref, g_ref, b_ref, bt_ref, p_ref):
    a, c = _bn_scale(sum_ref[...], ssq_ref[...], g_ref[...], b_ref[...])
    h = m_ref[...] * a + c
    brow = bt_ref[...].reshape(1, _BLK)
    oh = (lax.broadcasted_iota(jnp.int32, (N_GRAPHS, _BLK), 0)
          == brow).astype(jnp.float32)
    pp = jnp.dot(oh, h, preferred_element_type=jnp.float32)

    @pl.when(pl.program_id(0) == 0)
    def _():
        p_ref[...] = pp

    @pl.when(pl.program_id(0) != 0)
    def _():
        p_ref[...] += pp


def _bn_pool(m, s8, q8, g, b, batch3):
    return pl.pallas_call(
        _bn_pool_body,
        grid=(_NBLK,),
        in_specs=[_nodes_spec, _stat_spec, _stat_spec, _b_spec, _b_spec,
                  pl.BlockSpec((1, 1, _BLK), lambda i: (i, 0, 0))],
        out_specs=pl.BlockSpec((N_GRAPHS, DIM), lambda i: (0, 0)),
        out_shape=jax.ShapeDtypeStruct((N_GRAPHS, DIM), jnp.float32),
    )(m, s8, q8, g, b, batch3)


_HG = 8


def _heads_body(t2_ref, t1_ref, w2s_ref, a2t_ref, b2c_ref, w1s_ref, b1c_ref,
                c2_ref, c1_ref):
    t2 = t2_ref[...].reshape(_HG, 1000)
    t1 = t1_ref[...]
    for g in range(_HG):
        t2g = t2[g:g + 1, :]
        oh = (lax.broadcasted_iota(jnp.int32, (32, 1000), 0)
              == t2g).astype(jnp.float32)
        c2 = jnp.zeros((32, 121), jnp.float32)
        for k in range(8):
            bk = lax.dot_general(w2s_ref[k], oh, (((1,), (1,)), ((), ())),
                                 preferred_element_type=jnp.float32)
            c2 = c2 + jnp.dot(bk, a2t_ref[k],
                              preferred_element_type=jnp.float32)
        c2_ref[g] = c2 + b2c_ref[...]

        t1g = t1[g]
        c1 = jnp.zeros((32, 17), jnp.float32)
        for k in range(8):
            c1 = c1 + jnp.dot(w1s_ref[k], t1g[:, k:k + 17],
                              preferred_element_type=jnp.float32)
        c1_ref[g] = c1 + b1c_ref[...]


def _conv_heads(t2r, t1, w2s, a2t, b2c, w1s, b1c):
    return pl.pallas_call(
        _heads_body,
        grid=(N_GRAPHS // _HG,),
        in_specs=[
            pl.BlockSpec((_HG, 1, 1000), lambda i: (i, 0, 0)),
            pl.BlockSpec((_HG, 20, 24), lambda i: (i, 0, 0)),
            pl.BlockSpec((8, 32, 1000), lambda i: (0, 0, 0)),
            pl.BlockSpec((8, 32, 121), lambda i: (0, 0, 0)),
            pl.BlockSpec((32, 1), lambda i: (0, 0)),
            pl.BlockSpec((8, 32, 20), lambda i: (0, 0, 0)),
            pl.BlockSpec((32, 1), lambda i: (0, 0)),
        ],
        out_specs=(pl.BlockSpec((_HG, 32, 121), lambda i: (i, 0, 0)),
                   pl.BlockSpec((_HG, 32, 17), lambda i: (i, 0, 0))),
        out_shape=(jax.ShapeDtypeStruct((N_GRAPHS, 32, 121), jnp.float32),
                   jax.ShapeDtypeStruct((N_GRAPHS, 32, 17), jnp.float32)),
    )(t2r, t1, w2s, a2t, b2c, w1s, b1c)


def _final_body(p_ref, c1f_ref, c2f_ref, wxd_ref, bxd_ref, w1x_ref, b1x_ref,
                w2x_ref, b2x_ref, wa_ref, wb_ref, wc_ref, bf1_ref,
                wf2_ref, bf2_ref, wo_ref, bo_ref, o_ref):
    f32 = jnp.float32
    xd = jnp.maximum(
        jnp.dot(p_ref[...], wxd_ref[...], preferred_element_type=f32)
        + bxd_ref[...], 0.0)
    xt1 = jnp.dot(c1f_ref[...], w1x_ref[...],
                  preferred_element_type=f32) + b1x_ref[...]
    xt2 = jnp.dot(c2f_ref[...], w2x_ref[...],
                  preferred_element_type=f32) + b2x_ref[...]
    z = jnp.maximum(
        jnp.dot(xd, wa_ref[...], preferred_element_type=f32)
        + jnp.dot(xt1, wb_ref[...], preferred_element_type=f32)
        + jnp.dot(xt2, wc_ref[...], preferred_element_type=f32)
        + bf1_ref[...], 0.0)
    z2 = jnp.maximum(
        jnp.dot(z, wf2_ref[...], preferred_element_type=f32)
        + bf2_ref[...], 0.0)
    o_ref[...] = jnp.dot(z2, wo_ref[...],
                         preferred_element_type=f32) + bo_ref[...]


def _final_mlp(pooled, c1f, c2f, wxd, bxd, w1x, b1x, w2x, b2x,
               wa, wb, wc, bf1, wf2, bf2, wo, bo):
    return pl.pallas_call(
        _final_body,
        out_shape=jax.ShapeDtypeStruct((N_GRAPHS, 1), jnp.float32),
    )(pooled, c1f, c2f, wxd, bxd, w1x, b1x, w2x, b2x,
      wa, wb, wc, bf1, wf2, bf2, wo, bo)


def kernel(x, edge_index, batch, target1, target2, params):
    p = params
    src2 = edge_index[0].reshape(_CHUNKS, _CH)
    dst2 = edge_index[1].reshape(_CHUNKS, _CH)

    def row(v):
        return v.reshape(1, DIM)

    batch3 = batch.astype(jnp.int32).reshape(_NBLK, 1, _BLK)
    deg = _segsum_call(jnp.ones((N_NODES, DIM), jnp.float32), src2, dst2)
    y = _node_matmul(x, p['gin0_w1'])
    s = _segsum_call(y, src2, dst2)
    m, s8, q8 = _gin_mlp0(y, s, row(p['gin0_b1']),
                          p['gin0_w2'], row(p['gin0_b2']))
    for i in range(1, 5):
        s = _segsum_call(m, src2, dst2)
        m, s8, q8 = _gin_mlp(m, s, deg, s8, q8,
                             row(p['bn%d_g' % (i - 1)]),
                             row(p['bn%d_b' % (i - 1)]),
                             p['gin%d_w1' % i], row(p['gin%d_b1' % i]),
                             p['gin%d_w2' % i], row(p['gin%d_b2' % i]))
    pooled = _bn_pool(m, s8, q8, row(p['bn4_g']), row(p['bn4_b']), batch3)

    t2r = target2.astype(jnp.int32).reshape(N_GRAPHS, 1, 1000)
    epad = jnp.concatenate(
        [p['emb'], jnp.zeros((6, 128), jnp.float32)], axis=0)
    a2t = jnp.stack([epad[:, k:k + 121] for k in range(8)])
    w2s = jnp.transpose(p['cxt2_w'], (2, 0, 1))
    w1s = jnp.transpose(p['cxt1_w'], (2, 0, 1))
    c2, c1 = _conv_heads(t2r, target1, w2s, a2t,
                         p['cxt2_b'].reshape(32, 1), w1s,
                         p['cxt1_b'].reshape(32, 1))
    c2f = c2.reshape(N_GRAPHS, 32 * 121)
    c1f = c1.reshape(N_GRAPHS, 32 * 17)

    fw = p['fc1_w']
    out = _final_mlp(
        pooled, c1f, c2f,
        p['fc1_xd_w'], p['fc1_xd_b'].reshape(1, -1),
        p['fc1_xt_w'], p['fc1_xt_b'].reshape(1, -1),
        p['fc2_xt_w'], p['fc2_xt_b'].reshape(1, -1),
        fw[0:128], fw[128:256], fw[256:384], p['fc1_b'].reshape(1, -1),
        p['fc2_w'], p['fc2_b'].reshape(1, -1),
        p['out_w'], p['out_b'].reshape(1, -1))
    return out


# speedup vs baseline: 14.0169x; 1.2009x over previous
import jax
import jax.numpy as jnp
from jax import lax
from jax.experimental import pallas as pl
from jax.experimental.pallas import tpu as pltpu
from jax.experimental.pallas import tpu_sc as plsc

N_GRAPHS = 128
N_NODES = 50000
DIM = 32
N_EDGES = 800000

_NC, _NS = 2, 16
_CH = 128
_CHUNKS = N_EDGES // _CH
_CPC = _CHUNKS // _NC
_RPS = 3120
_ZR = 120
_NZ = _RPS // _ZR
_REM0 = _NS * _RPS
_REM = N_NODES - _REM0

_sc_mesh = plsc.VectorSubcoreMesh(
    core_axis_name="c", subcore_axis_name="s", num_cores=_NC, num_subcores=_NS)


def _edge_pipeline(y_hbm, acc, sidx4, didx4, bufs, gsems, ssems, n):
    gd = [None] * n
    sd = [None] * n
    for t in range(min(4, n)):
        gd[t] = pltpu.async_copy(y_hbm.at[sidx4.at[t]], bufs[t % 4],
                                 gsems[t % 4])
    for t in range(n):
        gd[t].wait()
        sd[t] = pltpu.async_copy(bufs[t % 4], acc.at[didx4.at[t]],
                                 ssems[t % 4], add=True)
        if t + 4 < n:
            sd[t].wait()
            gd[t + 4] = pltpu.async_copy(y_hbm.at[sidx4.at[t + 4]],
                                         bufs[t % 4], gsems[t % 4])
    for t in range(max(0, n - 4), n):
        sd[t].wait()


def _segsum_body(y_hbm, src2_hbm, dst2_hbm, out_hbm,
                 sidx4, didx4, buf0, buf1, buf2, buf3, zbuf, acc,
                 g0, g1, g2, g3, s0, s1, s2, s3):
    cid = lax.axis_index("c")
    sid = lax.axis_index("s")

    zv = jnp.zeros((16,), jnp.float32)

    def zb(i, carry):
        zbuf[i, pl.ds(0, 16)] = zv
        zbuf[i, pl.ds(16, 16)] = zv
        return carry

    lax.fori_loop(0, _ZR, zb, 0)

    def zacc(i, carry):
        pltpu.sync_copy(zbuf, acc.at[pl.ds(sid * _RPS + i * _ZR, _ZR)])
        return carry

    lax.fori_loop(0, _NZ, zacc, 0)

    @pl.when(sid == 0)
    def _():
        pltpu.sync_copy(zbuf.at[pl.ds(0, _REM)], acc.at[pl.ds(_REM0, _REM)])

    plsc.subcore_barrier()

    bufs = (buf0, buf1, buf2, buf3)
    gsems = (g0, g1, g2, g3)
    ssems = (s0, s1, s2, s3)
    extra = jnp.where(cid == 0, 2, 1)
    nb = 6 + jnp.where(sid < extra, 1, 0)
    core_base = cid * 98

    def sb_step(j, carry):
        row0 = (core_base + sid + _NS * j) * 32
        pltpu.sync_copy(src2_hbm.at[pl.ds(row0, 32)], sidx4)
        pltpu.sync_copy(dst2_hbm.at[pl.ds(row0, 32)], didx4)
        _edge_pipeline(y_hbm, acc, sidx4, didx4, bufs, gsems, ssems, 32)
        return carry

    lax.fori_loop(0, nb, sb_step, 0)

    @pl.when((cid == 1) & (sid == _NS - 1))
    def _():
        ntail = _CHUNKS - 195 * 32
        pltpu.sync_copy(src2_hbm.at[pl.ds(195 * 32, ntail)],
                        sidx4.at[pl.ds(0, ntail)])
        pltpu.sync_copy(dst2_hbm.at[pl.ds(195 * 32, ntail)],
                        didx4.at[pl.ds(0, ntail)])
        _edge_pipeline(y_hbm, acc, sidx4, didx4, bufs, gsems, ssems, ntail)

    plsc.subcore_barrier()

    def flush(i, carry):
        r0 = sid * _RPS + i * _ZR
        pltpu.sync_copy(acc.at[pl.ds(r0, _ZR)],
                        out_hbm.at[pl.ds(cid * N_NODES + r0, _ZR)])
        return carry

    lax.fori_loop(0, _NZ, flush, 0)

    @pl.when(sid == 0)
    def _():
        pltpu.sync_copy(acc.at[pl.ds(_REM0, _REM)],
                        out_hbm.at[pl.ds(cid * N_NODES + _REM0, _REM)])


_segsum_call = pl.kernel(
    _segsum_body,
    out_type=jax.ShapeDtypeStruct((_NC * N_NODES, DIM), jnp.float32),
    mesh=_sc_mesh,
    scratch_types=[
        pltpu.VMEM((32, _CH), jnp.int32),
        pltpu.VMEM((32, _CH), jnp.int32),
        pltpu.VMEM((_CH, DIM), jnp.float32),
        pltpu.VMEM((_CH, DIM), jnp.float32),
        pltpu.VMEM((_CH, DIM), jnp.float32),
        pltpu.VMEM((_CH, DIM), jnp.float32),
        pltpu.VMEM((_ZR, DIM), jnp.float32),
        pltpu.VMEM_SHARED((N_NODES, DIM), jnp.float32),
        pltpu.SemaphoreType.DMA,
        pltpu.SemaphoreType.DMA,
        pltpu.SemaphoreType.DMA,
        pltpu.SemaphoreType.DMA,
        pltpu.SemaphoreType.DMA,
        pltpu.SemaphoreType.DMA,
        pltpu.SemaphoreType.DMA,
        pltpu.SemaphoreType.DMA,
    ],
    compiler_params=pltpu.CompilerParams(use_tc_tiling_on_sc=False),
)


_BLK = 2000
_NBLK = N_NODES // _BLK


def _matmul_body(x_ref, w_ref, o_ref):
    o_ref[...] = jnp.dot(x_ref[...], w_ref[...],
                         preferred_element_type=jnp.float32)


def _node_matmul(x, w):
    fin = x.shape[1]
    return pl.pallas_call(
        _matmul_body,
        grid=(_NBLK,),
        in_specs=[pl.BlockSpec((_BLK, fin), lambda i: (i, 0)),
                  pl.BlockSpec((fin, DIM), lambda i: (0, 0))],
        out_specs=pl.BlockSpec((_BLK, DIM), lambda i: (i, 0)),
        out_shape=jax.ShapeDtypeStruct((N_NODES, DIM), jnp.float32),
    )(x, w)


def _stats_accum(m, sum_ref, ssq_ref):
    ones = jnp.ones((8, m.shape[0]), jnp.float32)
    ps = jnp.dot(ones, m, preferred_element_type=jnp.float32)
    pq = jnp.dot(ones, m * m, preferred_element_type=jnp.float32)

    @pl.when(pl.program_id(0) == 0)
    def _():
        sum_ref[...] = ps
        ssq_ref[...] = pq

    @pl.when(pl.program_id(0) != 0)
    def _():
        sum_ref[...] += ps
        ssq_ref[...] += pq


def _gin0_body(y_ref, s0_ref, s1_ref, b1_ref, w2_ref, b2_ref,
               m_ref, sum_ref, ssq_ref):
    u = jnp.maximum(y_ref[...] + s0_ref[...] + s1_ref[...] + b1_ref[...], 0.0)
    m = jnp.maximum(
        jnp.dot(u, w2_ref[...], preferred_element_type=jnp.float32)
        + b2_ref[...], 0.0)
    m_ref[...] = m
    _stats_accum(m, sum_ref, ssq_ref)


def _gin_body(m_ref, s0_ref, s1_ref, d0_ref, d1_ref, sum_ref, ssq_ref,
              g_ref, b_ref, w1_ref, b1_ref, w2_ref, b2_ref,
              mo_ref, sumo_ref, ssqo_ref):
    a, c = _bn_scale(sum_ref[...], ssq_ref[...], g_ref[...], b_ref[...])
    t = (a * (m_ref[...] + s0_ref[...] + s1_ref[...])
         + c * (1.0 + d0_ref[...] + d1_ref[...]))
    u = jnp.maximum(
        jnp.dot(t, w1_ref[...], preferred_element_type=jnp.float32)
        + b1_ref[...], 0.0)
    m = jnp.maximum(
        jnp.dot(u, w2_ref[...], preferred_element_type=jnp.float32)
        + b2_ref[...], 0.0)
    mo_ref[...] = m
    _stats_accum(m, sumo_ref, ssqo_ref)


_nodes_spec = pl.BlockSpec((_BLK, DIM), lambda i: (i, 0))
_part0_spec = pl.BlockSpec((_BLK, DIM), lambda i: (i, 0))
_part1_spec = pl.BlockSpec((_BLK, DIM), lambda i: (i + _NBLK, 0))
_w_spec = pl.BlockSpec((DIM, DIM), lambda i: (0, 0))
_b_spec = pl.BlockSpec((1, DIM), lambda i: (0, 0))
_stat_spec = pl.BlockSpec((8, DIM), lambda i: (0, 0))
_mlp_out_shapes = (
    jax.ShapeDtypeStruct((N_NODES, DIM), jnp.float32),
    jax.ShapeDtypeStruct((8, DIM), jnp.float32),
    jax.ShapeDtypeStruct((8, DIM), jnp.float32),
)


def _gin_mlp0(y, s, b1, w2, b2):
    return pl.pallas_call(
        _gin0_body,
        grid=(_NBLK,),
        in_specs=[_nodes_spec, _part0_spec, _part1_spec,
                  _b_spec, _w_spec, _b_spec],
        out_specs=(_nodes_spec, _stat_spec, _stat_spec),
        out_shape=_mlp_out_shapes,
    )(y, s, s, b1, w2, b2)


def _gin_mlp(m, s, d, s8, q8, g, b, w1, b1, w2, b2):
    return pl.pallas_call(
        _gin_body,
        grid=(_NBLK,),
        in_specs=[_nodes_spec, _part0_spec, _part1_spec,
                  _part0_spec, _part1_spec,
                  _stat_spec, _stat_spec, _b_spec, _b_spec,
                  _w_spec, _b_spec, _w_spec, _b_spec],
        out_specs=(_nodes_spec, _stat_spec, _stat_spec),
        out_shape=_mlp_out_shapes,
    )(m, s, s, d, d, s8, q8, g, b, w1, b1, w2, b2)


def _bn_scale(sum_v, ssq_v, g_v, b_v):
    s = jnp.sum(sum_v, axis=0, keepdims=True)
    q = jnp.sum(ssq_v, axis=0, keepdims=True)
    mu = s * (1.0 / N_NODES)
    var = q * (1.0 / N_NODES) - mu * mu
    a = g_v * lax.rsqrt(var + 1e-5)
    c = b_v - mu * a
    return a, c


def _bn_pool_body(m_ref, sum_ref, ssq_ref, g_ref, b_ref, bt_ref, p_ref):
    a, c = _bn_scale(sum_ref[...], ssq_ref[...], g_ref[...], b_ref[...])
    h = m_ref[...] * a + c
    brow = bt_ref[...].reshape(1, _BLK)
    oh = (lax.broadcasted_iota(jnp.int32, (N_GRAPHS, _BLK), 0)
          == brow).astype(jnp.float32)
    pp = jnp.dot(oh, h, preferred_element_type=jnp.float32)

    @pl.when(pl.program_id(0) == 0)
    def _():
        p_ref[...] = pp

    @pl.when(pl.program_id(0) != 0)
    def _():
        p_ref[...] += pp


def _bn_pool(m, s8, q8, g, b, batch3):
    return pl.pallas_call(
        _bn_pool_body,
        grid=(_NBLK,),
        in_specs=[_nodes_spec, _stat_spec, _stat_spec, _b_spec, _b_spec,
                  pl.BlockSpec((1, 1, _BLK), lambda i: (i, 0, 0))],
        out_specs=pl.BlockSpec((N_GRAPHS, DIM), lambda i: (0, 0)),
        out_shape=jax.ShapeDtypeStruct((N_GRAPHS, DIM), jnp.float32),
    )(m, s8, q8, g, b, batch3)


_HG = 8


def _heads_body(t2_ref, t1_ref, w2s_ref, a2t_ref, b2c_ref, w1s_ref, b1c_ref,
                c2_ref, c1_ref):
    t2 = t2_ref[...].reshape(_HG, 1000)
    t1 = t1_ref[...]
    for g in range(_HG):
        t2g = t2[g:g + 1, :]
        oh = (lax.broadcasted_iota(jnp.int32, (32, 1000), 0)
              == t2g).astype(jnp.float32)
        c2 = jnp.zeros((32, 121), jnp.float32)
        for k in range(8):
            bk = lax.dot_general(w2s_ref[k], oh, (((1,), (1,)), ((), ())),
                                 preferred_element_type=jnp.float32)
            c2 = c2 + jnp.dot(bk, a2t_ref[k],
                              preferred_element_type=jnp.float32)
        c2_ref[g] = c2 + b2c_ref[...]

        t1g = t1[g]
        c1 = jnp.zeros((32, 17), jnp.float32)
        for k in range(8):
            c1 = c1 + jnp.dot(w1s_ref[k], t1g[:, k:k + 17],
                              preferred_element_type=jnp.float32)
        c1_ref[g] = c1 + b1c_ref[...]


def _conv_heads(t2r, t1, w2s, a2t, b2c, w1s, b1c):
    return pl.pallas_call(
        _heads_body,
        grid=(N_GRAPHS // _HG,),
        in_specs=[
            pl.BlockSpec((_HG, 1, 1000), lambda i: (i, 0, 0)),
            pl.BlockSpec((_HG, 20, 24), lambda i: (i, 0, 0)),
            pl.BlockSpec((8, 32, 1000), lambda i: (0, 0, 0)),
            pl.BlockSpec((8, 32, 121), lambda i: (0, 0, 0)),
            pl.BlockSpec((32, 1), lambda i: (0, 0)),
            pl.BlockSpec((8, 32, 20), lambda i: (0, 0, 0)),
            pl.BlockSpec((32, 1), lambda i: (0, 0)),
        ],
        out_specs=(pl.BlockSpec((_HG, 32, 121), lambda i: (i, 0, 0)),
                   pl.BlockSpec((_HG, 32, 17), lambda i: (i, 0, 0))),
        out_shape=(jax.ShapeDtypeStruct((N_GRAPHS, 32, 121), jnp.float32),
                   jax.ShapeDtypeStruct((N_GRAPHS, 32, 17), jnp.float32)),
    )(t2r, t1, w2s, a2t, b2c, w1s, b1c)


def _final_body(p_ref, c1f_ref, c2f_ref, wxd_ref, bxd_ref, w1x_ref, b1x_ref,
                w2x_ref, b2x_ref, wa_ref, wb_ref, wc_ref, bf1_ref,
                wf2_ref, bf2_ref, wo_ref, bo_ref, o_ref):
    f32 = jnp.float32
    xd = jnp.maximum(
        jnp.dot(p_ref[...], wxd_ref[...], preferred_element_type=f32)
        + bxd_ref[...], 0.0)
    xt1 = jnp.dot(c1f_ref[...], w1x_ref[...],
                  preferred_element_type=f32) + b1x_ref[...]
    xt2 = jnp.dot(c2f_ref[...], w2x_ref[...],
                  preferred_element_type=f32) + b2x_ref[...]
    z = jnp.maximum(
        jnp.dot(xd, wa_ref[...], preferred_element_type=f32)
        + jnp.dot(xt1, wb_ref[...], preferred_element_type=f32)
        + jnp.dot(xt2, wc_ref[...], preferred_element_type=f32)
        + bf1_ref[...], 0.0)
    z2 = jnp.maximum(
        jnp.dot(z, wf2_ref[...], preferred_element_type=f32)
        + bf2_ref[...], 0.0)
    o_ref[...] = jnp.dot(z2, wo_ref[...],
                         preferred_element_type=f32) + bo_ref[...]


def _final_mlp(pooled, c1f, c2f, wxd, bxd, w1x, b1x, w2x, b2x,
               wa, wb, wc, bf1, wf2, bf2, wo, bo):
    return pl.pallas_call(
        _final_body,
        out_shape=jax.ShapeDtypeStruct((N_GRAPHS, 1), jnp.float32),
    )(pooled, c1f, c2f, wxd, bxd, w1x, b1x, w2x, b2x,
      wa, wb, wc, bf1, wf2, bf2, wo, bo)


def kernel(x, edge_index, batch, target1, target2, params):
    p = params
    src2 = edge_index[0].reshape(_CHUNKS, _CH)
    dst2 = edge_index[1].reshape(_CHUNKS, _CH)

    def row(v):
        return v.reshape(1, DIM)

    batch3 = batch.astype(jnp.int32).reshape(_NBLK, 1, _BLK)
    deg = _segsum_call(jnp.ones((N_NODES, DIM), jnp.float32), src2, dst2)
    y = _node_matmul(x, p['gin0_w1'])
    s = _segsum_call(y, src2, dst2)
    m, s8, q8 = _gin_mlp0(y, s, row(p['gin0_b1']),
                          p['gin0_w2'], row(p['gin0_b2']))
    for i in range(1, 5):
        s = _segsum_call(m, src2, dst2)
        m, s8, q8 = _gin_mlp(m, s, deg, s8, q8,
                             row(p['bn%d_g' % (i - 1)]),
                             row(p['bn%d_b' % (i - 1)]),
                             p['gin%d_w1' % i], row(p['gin%d_b1' % i]),
                             p['gin%d_w2' % i], row(p['gin%d_b2' % i]))
    pooled = _bn_pool(m, s8, q8, row(p['bn4_g']), row(p['bn4_b']), batch3)

    t2r = target2.astype(jnp.int32).reshape(N_GRAPHS, 1, 1000)
    epad = jnp.concatenate(
        [p['emb'], jnp.zeros((6, 128), jnp.float32)], axis=0)
    a2t = jnp.stack([epad[:, k:k + 121] for k in range(8)])
    w2s = jnp.transpose(p['cxt2_w'], (2, 0, 1))
    w1s = jnp.transpose(p['cxt1_w'], (2, 0, 1))
    c2, c1 = _conv_heads(t2r, target1, w2s, a2t,
                         p['cxt2_b'].reshape(32, 1), w1s,
                         p['cxt1_b'].reshape(32, 1))
    c2f = c2.reshape(N_GRAPHS, 32 * 121)
    c1f = c1.reshape(N_GRAPHS, 32 * 17)

    fw = p['fc1_w']
    out = _final_mlp(
        pooled, c1f, c2f,
        p['fc1_xd_w'], p['fc1_xd_b'].reshape(1, -1),
        p['fc1_xt_w'], p['fc1_xt_b'].reshape(1, -1),
        p['fc2_xt_w'], p['fc2_xt_b'].reshape(1, -1),
        fw[0:128], fw[128:256], fw[256:384], p['fc1_b'].reshape(1, -1),
        p['fc2_w'], p['fc2_b'].reshape(1, -1),
        p['out_w'], p['out_b'].reshape(1, -1))
    return out


# speedup vs baseline: 14.5315x; 1.0367x over previous
import jax
import jax.numpy as jnp
from jax import lax
from jax.experimental import pallas as pl
from jax.experimental.pallas import tpu as pltpu
from jax.experimental.pallas import tpu_sc as plsc

N_GRAPHS = 128
N_NODES = 50000
DIM = 32
N_EDGES = 800000

_NC, _NS = 2, 16
_CH = 128
_CHUNKS = N_EDGES // _CH
_CPC = _CHUNKS // _NC
_RPS = 3120
_ZR = 120
_NZ = _RPS // _ZR
_REM0 = _NS * _RPS
_REM = N_NODES - _REM0

_sc_mesh = plsc.VectorSubcoreMesh(
    core_axis_name="c", subcore_axis_name="s", num_cores=_NC, num_subcores=_NS)


def _edge_pipeline(y_hbm, acc, sidx4, didx4, bufs, gsems, ssems, n):
    gd = [None] * n
    sd = [None] * n
    for t in range(min(4, n)):
        gd[t] = pltpu.async_copy(y_hbm.at[sidx4.at[t]], bufs[t % 4],
                                 gsems[t % 4])
    for t in range(n):
        gd[t].wait()
        sd[t] = pltpu.async_copy(bufs[t % 4], acc.at[didx4.at[t]],
                                 ssems[t % 4], add=True)
        if t + 4 < n:
            sd[t].wait()
            gd[t + 4] = pltpu.async_copy(y_hbm.at[sidx4.at[t + 4]],
                                         bufs[t % 4], gsems[t % 4])
    for t in range(max(0, n - 4), n):
        sd[t].wait()


def _segsum_body(y_hbm, src2_hbm, dst2_hbm, out_hbm,
                 sidx4, didx4, buf0, buf1, buf2, buf3, zbuf, acc,
                 g0, g1, g2, g3, s0, s1, s2, s3):
    cid = lax.axis_index("c")
    sid = lax.axis_index("s")

    zv = jnp.zeros((16,), jnp.float32)

    def zb(i, carry):
        zbuf[i, pl.ds(0, 16)] = zv
        zbuf[i, pl.ds(16, 16)] = zv
        return carry

    lax.fori_loop(0, _ZR, zb, 0)

    def zacc(i, carry):
        pltpu.sync_copy(zbuf, acc.at[pl.ds(sid * _RPS + i * _ZR, _ZR)])
        return carry

    lax.fori_loop(0, _NZ, zacc, 0)

    @pl.when(sid == 0)
    def _():
        pltpu.sync_copy(zbuf.at[pl.ds(0, _REM)], acc.at[pl.ds(_REM0, _REM)])

    plsc.subcore_barrier()

    bufs = (buf0, buf1, buf2, buf3)
    gsems = (g0, g1, g2, g3)
    ssems = (s0, s1, s2, s3)
    extra = jnp.where(cid == 0, 2, 1)
    nb = 6 + jnp.where(sid < extra, 1, 0)
    core_base = cid * 98

    def sb_step(j, carry):
        row0 = (core_base + sid + _NS * j) * 32
        pltpu.sync_copy(src2_hbm.at[pl.ds(row0, 32)], sidx4)
        pltpu.sync_copy(dst2_hbm.at[pl.ds(row0, 32)], didx4)
        _edge_pipeline(y_hbm, acc, sidx4, didx4, bufs, gsems, ssems, 32)
        return carry

    lax.fori_loop(0, nb, sb_step, 0)

    @pl.when((cid == 1) & (sid == _NS - 1))
    def _():
        ntail = _CHUNKS - 195 * 32
        pltpu.sync_copy(src2_hbm.at[pl.ds(195 * 32, ntail)],
                        sidx4.at[pl.ds(0, ntail)])
        pltpu.sync_copy(dst2_hbm.at[pl.ds(195 * 32, ntail)],
                        didx4.at[pl.ds(0, ntail)])
        _edge_pipeline(y_hbm, acc, sidx4, didx4, bufs, gsems, ssems, ntail)

    plsc.subcore_barrier()

    def flush(i, carry):
        r0 = sid * _RPS + i * _ZR
        pltpu.sync_copy(acc.at[pl.ds(r0, _ZR)],
                        out_hbm.at[pl.ds(cid * N_NODES + r0, _ZR)])
        return carry

    lax.fori_loop(0, _NZ, flush, 0)

    @pl.when(sid == 0)
    def _():
        pltpu.sync_copy(acc.at[pl.ds(_REM0, _REM)],
                        out_hbm.at[pl.ds(cid * N_NODES + _REM0, _REM)])


def _deg_body(dst2_hbm, out_hbm, didx4, ones_buf, zbuf, acc, s0, s1, s2, s3):
    cid = lax.axis_index("c")
    sid = lax.axis_index("s")
    ssems = (s0, s1, s2, s3)

    zero = jnp.zeros((16,), jnp.float32)
    one = jnp.ones((16,), jnp.float32)

    def fb(i, carry):
        zbuf[i, pl.ds(0, 16)] = zero
        zbuf[i, pl.ds(16, 16)] = zero
        return carry

    lax.fori_loop(0, _ZR, fb, 0)

    def ob(i, carry):
        ones_buf[i, pl.ds(0, 16)] = one
        ones_buf[i, pl.ds(16, 16)] = one
        return carry

    lax.fori_loop(0, _CH, ob, 0)

    def zacc(i, carry):
        pltpu.sync_copy(zbuf, acc.at[pl.ds(sid * _RPS + i * _ZR, _ZR)])
        return carry

    lax.fori_loop(0, _NZ, zacc, 0)

    @pl.when(sid == 0)
    def _():
        pltpu.sync_copy(zbuf.at[pl.ds(0, _REM)], acc.at[pl.ds(_REM0, _REM)])

    plsc.subcore_barrier()

    extra = jnp.where(cid == 0, 2, 1)
    nb = 6 + jnp.where(sid < extra, 1, 0)
    core_base = cid * 98

    def scatter_run(n):
        sd = [None] * n
        for t in range(n):
            if t >= 8:
                sd[t - 8].wait()
            sd[t] = pltpu.async_copy(ones_buf, acc.at[didx4.at[t]],
                                     ssems[t % 4], add=True)
        for t in range(max(0, n - 8), n):
            sd[t].wait()

    def sb_step(j, carry):
        row0 = (core_base + sid + _NS * j) * 32
        pltpu.sync_copy(dst2_hbm.at[pl.ds(row0, 32)], didx4)
        scatter_run(32)
        return carry

    lax.fori_loop(0, nb, sb_step, 0)

    @pl.when((cid == 1) & (sid == _NS - 1))
    def _():
        ntail = _CHUNKS - 195 * 32
        pltpu.sync_copy(dst2_hbm.at[pl.ds(195 * 32, ntail)],
                        didx4.at[pl.ds(0, ntail)])
        scatter_run(ntail)

    plsc.subcore_barrier()

    def flush(i, carry):
        r0 = sid * _RPS + i * _ZR
        pltpu.sync_copy(acc.at[pl.ds(r0, _ZR)],
                        out_hbm.at[pl.ds(cid * N_NODES + r0, _ZR)])
        return carry

    lax.fori_loop(0, _NZ, flush, 0)

    @pl.when(sid == 0)
    def _():
        pltpu.sync_copy(acc.at[pl.ds(_REM0, _REM)],
                        out_hbm.at[pl.ds(cid * N_NODES + _REM0, _REM)])


_deg_call = pl.kernel(
    _deg_body,
    out_type=jax.ShapeDtypeStruct((_NC * N_NODES, DIM), jnp.float32),
    mesh=_sc_mesh,
    scratch_types=[
        pltpu.VMEM((32, _CH), jnp.int32),
        pltpu.VMEM((_CH, DIM), jnp.float32),
        pltpu.VMEM((_ZR, DIM), jnp.float32),
        pltpu.VMEM_SHARED((N_NODES, DIM), jnp.float32),
        pltpu.SemaphoreType.DMA,
        pltpu.SemaphoreType.DMA,
        pltpu.SemaphoreType.DMA,
        pltpu.SemaphoreType.DMA,
    ],
    compiler_params=pltpu.CompilerParams(use_tc_tiling_on_sc=False),
)


_segsum_call = pl.kernel(
    _segsum_body,
    out_type=jax.ShapeDtypeStruct((_NC * N_NODES, DIM), jnp.float32),
    mesh=_sc_mesh,
    scratch_types=[
        pltpu.VMEM((32, _CH), jnp.int32),
        pltpu.VMEM((32, _CH), jnp.int32),
        pltpu.VMEM((_CH, DIM), jnp.float32),
        pltpu.VMEM((_CH, DIM), jnp.float32),
        pltpu.VMEM((_CH, DIM), jnp.float32),
        pltpu.VMEM((_CH, DIM), jnp.float32),
        pltpu.VMEM((_ZR, DIM), jnp.float32),
        pltpu.VMEM_SHARED((N_NODES, DIM), jnp.float32),
        pltpu.SemaphoreType.DMA,
        pltpu.SemaphoreType.DMA,
        pltpu.SemaphoreType.DMA,
        pltpu.SemaphoreType.DMA,
        pltpu.SemaphoreType.DMA,
        pltpu.SemaphoreType.DMA,
        pltpu.SemaphoreType.DMA,
        pltpu.SemaphoreType.DMA,
    ],
    compiler_params=pltpu.CompilerParams(use_tc_tiling_on_sc=False),
)


_BLK = 2000
_NBLK = N_NODES // _BLK


def _matmul_body(x_ref, w_ref, o_ref):
    o_ref[...] = jnp.dot(x_ref[...], w_ref[...],
                         preferred_element_type=jnp.float32)


def _node_matmul(x, w):
    fin = x.shape[1]
    return pl.pallas_call(
        _matmul_body,
        grid=(_NBLK,),
        in_specs=[pl.BlockSpec((_BLK, fin), lambda i: (i, 0)),
                  pl.BlockSpec((fin, DIM), lambda i: (0, 0))],
        out_specs=pl.BlockSpec((_BLK, DIM), lambda i: (i, 0)),
        out_shape=jax.ShapeDtypeStruct((N_NODES, DIM), jnp.float32),
    )(x, w)


def _stats_accum(m, sum_ref, ssq_ref):
    ones = jnp.ones((8, m.shape[0]), jnp.float32)
    ps = jnp.dot(ones, m, preferred_element_type=jnp.float32)
    pq = jnp.dot(ones, m * m, preferred_element_type=jnp.float32)

    @pl.when(pl.program_id(0) == 0)
    def _():
        sum_ref[...] = ps
        ssq_ref[...] = pq

    @pl.when(pl.program_id(0) != 0)
    def _():
        sum_ref[...] += ps
        ssq_ref[...] += pq


def _gin0_body(y_ref, s0_ref, s1_ref, b1_ref, w2_ref, b2_ref,
               m_ref, sum_ref, ssq_ref):
    u = jnp.maximum(y_ref[...] + s0_ref[...] + s1_ref[...] + b1_ref[...], 0.0)
    m = jnp.maximum(
        jnp.dot(u, w2_ref[...], preferred_element_type=jnp.float32)
        + b2_ref[...], 0.0)
    m_ref[...] = m
    _stats_accum(m, sum_ref, ssq_ref)


def _gin_body(m_ref, s0_ref, s1_ref, d0_ref, d1_ref, sum_ref, ssq_ref,
              g_ref, b_ref, w1_ref, b1_ref, w2_ref, b2_ref,
              mo_ref, sumo_ref, ssqo_ref):
    a, c = _bn_scale(sum_ref[...], ssq_ref[...], g_ref[...], b_ref[...])
    t = (a * (m_ref[...] + s0_ref[...] + s1_ref[...])
         + c * (1.0 + d0_ref[...] + d1_ref[...]))
    u = jnp.maximum(
        jnp.dot(t, w1_ref[...], preferred_element_type=jnp.float32)
        + b1_ref[...], 0.0)
    m = jnp.maximum(
        jnp.dot(u, w2_ref[...], preferred_element_type=jnp.float32)
        + b2_ref[...], 0.0)
    mo_ref[...] = m
    _stats_accum(m, sumo_ref, ssqo_ref)


_nodes_spec = pl.BlockSpec((_BLK, DIM), lambda i: (i, 0))
_part0_spec = pl.BlockSpec((_BLK, DIM), lambda i: (i, 0))
_part1_spec = pl.BlockSpec((_BLK, DIM), lambda i: (i + _NBLK, 0))
_w_spec = pl.BlockSpec((DIM, DIM), lambda i: (0, 0))
_b_spec = pl.BlockSpec((1, DIM), lambda i: (0, 0))
_stat_spec = pl.BlockSpec((8, DIM), lambda i: (0, 0))
_mlp_out_shapes = (
    jax.ShapeDtypeStruct((N_NODES, DIM), jnp.float32),
    jax.ShapeDtypeStruct((8, DIM), jnp.float32),
    jax.ShapeDtypeStruct((8, DIM), jnp.float32),
)


def _gin_mlp0(y, s, b1, w2, b2):
    return pl.pallas_call(
        _gin0_body,
        grid=(_NBLK,),
        in_specs=[_nodes_spec, _part0_spec, _part1_spec,
                  _b_spec, _w_spec, _b_spec],
        out_specs=(_nodes_spec, _stat_spec, _stat_spec),
        out_shape=_mlp_out_shapes,
    )(y, s, s, b1, w2, b2)


def _gin_mlp(m, s, d, s8, q8, g, b, w1, b1, w2, b2):
    return pl.pallas_call(
        _gin_body,
        grid=(_NBLK,),
        in_specs=[_nodes_spec, _part0_spec, _part1_spec,
                  _part0_spec, _part1_spec,
                  _stat_spec, _stat_spec, _b_spec, _b_spec,
                  _w_spec, _b_spec, _w_spec, _b_spec],
        out_specs=(_nodes_spec, _stat_spec, _stat_spec),
        out_shape=_mlp_out_shapes,
    )(m, s, s, d, d, s8, q8, g, b, w1, b1, w2, b2)


def _bn_scale(sum_v, ssq_v, g_v, b_v):
    s = jnp.sum(sum_v, axis=0, keepdims=True)
    q = jnp.sum(ssq_v, axis=0, keepdims=True)
    mu = s * (1.0 / N_NODES)
    var = q * (1.0 / N_NODES) - mu * mu
    a = g_v * lax.rsqrt(var + 1e-5)
    c = b_v - mu * a
    return a, c


def _bn_pool_body(m_ref, sum_ref, ssq_ref, g_ref, b_ref, bt_ref, p_ref):
    a, c = _bn_scale(sum_ref[...], ssq_ref[...], g_ref[...], b_ref[...])
    h = m_ref[...] * a + c
    brow = bt_ref[...].reshape(1, _BLK)
    oh = (lax.broadcasted_iota(jnp.int32, (N_GRAPHS, _BLK), 0)
          == brow).astype(jnp.float32)
    pp = jnp.dot(oh, h, preferred_element_type=jnp.float32)

    @pl.when(pl.program_id(0) == 0)
    def _():
        p_ref[...] = pp

    @pl.when(pl.program_id(0) != 0)
    def _():
        p_ref[...] += pp


def _bn_pool(m, s8, q8, g, b, batch3):
    return pl.pallas_call(
        _bn_pool_body,
        grid=(_NBLK,),
        in_specs=[_nodes_spec, _stat_spec, _stat_spec, _b_spec, _b_spec,
                  pl.BlockSpec((1, 1, _BLK), lambda i: (i, 0, 0))],
        out_specs=pl.BlockSpec((N_GRAPHS, DIM), lambda i: (0, 0)),
        out_shape=jax.ShapeDtypeStruct((N_GRAPHS, DIM), jnp.float32),
    )(m, s8, q8, g, b, batch3)


_HG = 8


def _heads_body(t2_ref, t1_ref, w2s_ref, a2t_ref, b2c_ref, w1s_ref, b1c_ref,
                c2_ref, c1_ref):
    t2 = t2_ref[...].reshape(_HG, 1000)
    t1 = t1_ref[...]
    for g in range(_HG):
        t2g = t2[g:g + 1, :]
        oh = (lax.broadcasted_iota(jnp.int32, (32, 1000), 0)
              == t2g).astype(jnp.float32)
        c2 = jnp.zeros((32, 121), jnp.float32)
        for k in range(8):
            bk = lax.dot_general(w2s_ref[k], oh, (((1,), (1,)), ((), ())),
                                 preferred_element_type=jnp.float32)
            c2 = c2 + jnp.dot(bk, a2t_ref[k],
                              preferred_element_type=jnp.float32)
        c2_ref[g] = c2 + b2c_ref[...]

        t1g = t1[g]
        c1 = jnp.zeros((32, 17), jnp.float32)
        for k in range(8):
            c1 = c1 + jnp.dot(w1s_ref[k], t1g[:, k:k + 17],
                              preferred_element_type=jnp.float32)
        c1_ref[g] = c1 + b1c_ref[...]


def _conv_heads(t2r, t1, w2s, a2t, b2c, w1s, b1c):
    return pl.pallas_call(
        _heads_body,
        grid=(N_GRAPHS // _HG,),
        in_specs=[
            pl.BlockSpec((_HG, 1, 1000), lambda i: (i, 0, 0)),
            pl.BlockSpec((_HG, 20, 24), lambda i: (i, 0, 0)),
            pl.BlockSpec((8, 32, 1000), lambda i: (0, 0, 0)),
            pl.BlockSpec((8, 32, 121), lambda i: (0, 0, 0)),
            pl.BlockSpec((32, 1), lambda i: (0, 0)),
            pl.BlockSpec((8, 32, 20), lambda i: (0, 0, 0)),
            pl.BlockSpec((32, 1), lambda i: (0, 0)),
        ],
        out_specs=(pl.BlockSpec((_HG, 32, 121), lambda i: (i, 0, 0)),
                   pl.BlockSpec((_HG, 32, 17), lambda i: (i, 0, 0))),
        out_shape=(jax.ShapeDtypeStruct((N_GRAPHS, 32, 121), jnp.float32),
                   jax.ShapeDtypeStruct((N_GRAPHS, 32, 17), jnp.float32)),
    )(t2r, t1, w2s, a2t, b2c, w1s, b1c)


def _final_body(p_ref, c1f_ref, c2f_ref, wxd_ref, bxd_ref, w1x_ref, b1x_ref,
                w2x_ref, b2x_ref, wa_ref, wb_ref, wc_ref, bf1_ref,
                wf2_ref, bf2_ref, wo_ref, bo_ref, o_ref):
    f32 = jnp.float32
    xd = jnp.maximum(
        jnp.dot(p_ref[...], wxd_ref[...], preferred_element_type=f32)
        + bxd_ref[...], 0.0)
    xt1 = jnp.dot(c1f_ref[...], w1x_ref[...],
                  preferred_element_type=f32) + b1x_ref[...]
    xt2 = jnp.dot(c2f_ref[...], w2x_ref[...],
                  preferred_element_type=f32) + b2x_ref[...]
    z = jnp.maximum(
        jnp.dot(xd, wa_ref[...], preferred_element_type=f32)
        + jnp.dot(xt1, wb_ref[...], preferred_element_type=f32)
        + jnp.dot(xt2, wc_ref[...], preferred_element_type=f32)
        + bf1_ref[...], 0.0)
    z2 = jnp.maximum(
        jnp.dot(z, wf2_ref[...], preferred_element_type=f32)
        + bf2_ref[...], 0.0)
    o_ref[...] = jnp.dot(z2, wo_ref[...],
                         preferred_element_type=f32) + bo_ref[...]


def _final_mlp(pooled, c1f, c2f, wxd, bxd, w1x, b1x, w2x, b2x,
               wa, wb, wc, bf1, wf2, bf2, wo, bo):
    return pl.pallas_call(
        _final_body,
        out_shape=jax.ShapeDtypeStruct((N_GRAPHS, 1), jnp.float32),
    )(pooled, c1f, c2f, wxd, bxd, w1x, b1x, w2x, b2x,
      wa, wb, wc, bf1, wf2, bf2, wo, bo)


def kernel(x, edge_index, batch, target1, target2, params):
    p = params
    src2 = edge_index[0].reshape(_CHUNKS, _CH)
    dst2 = edge_index[1].reshape(_CHUNKS, _CH)

    def row(v):
        return v.reshape(1, DIM)

    batch3 = batch.astype(jnp.int32).reshape(_NBLK, 1, _BLK)
    deg = _deg_call(dst2)
    y = _node_matmul(x, p['gin0_w1'])
    s = _segsum_call(y, src2, dst2)
    m, s8, q8 = _gin_mlp0(y, s, row(p['gin0_b1']),
                          p['gin0_w2'], row(p['gin0_b2']))
    for i in range(1, 5):
        s = _segsum_call(m, src2, dst2)
        m, s8, q8 = _gin_mlp(m, s, deg, s8, q8,
                             row(p['bn%d_g' % (i - 1)]),
                             row(p['bn%d_b' % (i - 1)]),
                             p['gin%d_w1' % i], row(p['gin%d_b1' % i]),
                             p['gin%d_w2' % i], row(p['gin%d_b2' % i]))
    pooled = _bn_pool(m, s8, q8, row(p['bn4_g']), row(p['bn4_b']), batch3)

    t2r = target2.astype(jnp.int32).reshape(N_GRAPHS, 1, 1000)
    epad = jnp.concatenate(
        [p['emb'], jnp.zeros((6, 128), jnp.float32)], axis=0)
    a2t = jnp.stack([epad[:, k:k + 121] for k in range(8)])
    w2s = jnp.transpose(p['cxt2_w'], (2, 0, 1))
    w1s = jnp.transpose(p['cxt1_w'], (2, 0, 1))
    c2, c1 = _conv_heads(t2r, target1, w2s, a2t,
                         p['cxt2_b'].reshape(32, 1), w1s,
                         p['cxt1_b'].reshape(32, 1))
    c2f = c2.reshape(N_GRAPHS, 32 * 121)
    c1f = c1.reshape(N_GRAPHS, 32 * 17)

    fw = p['fc1_w']
    out = _final_mlp(
        pooled, c1f, c2f,
        p['fc1_xd_w'], p['fc1_xd_b'].reshape(1, -1),
        p['fc1_xt_w'], p['fc1_xt_b'].reshape(1, -1),
        p['fc2_xt_w'], p['fc2_xt_b'].reshape(1, -1),
        fw[0:128], fw[128:256], fw[256:384], p['fc1_b'].reshape(1, -1),
        p['fc2_w'], p['fc2_b'].reshape(1, -1),
        p['out_w'], p['out_b'].reshape(1, -1))
    return out


# speedup vs baseline: 14.8690x; 1.0232x over previous
import jax
import jax.numpy as jnp
from jax import lax
from jax.experimental import pallas as pl
from jax.experimental.pallas import tpu as pltpu
from jax.experimental.pallas import tpu_sc as plsc

N_GRAPHS = 128
N_NODES = 50000
DIM = 32
N_EDGES = 800000

_NC, _NS = 2, 16
_CH = 128
_CHUNKS = N_EDGES // _CH
_CPC = _CHUNKS // _NC
_RPS = 3120
_ZR = 120
_NZ = _RPS // _ZR
_REM0 = _NS * _RPS
_REM = N_NODES - _REM0

_sc_mesh = plsc.VectorSubcoreMesh(
    core_axis_name="c", subcore_axis_name="s", num_cores=_NC, num_subcores=_NS)


def _edge_pipeline(y_hbm, acc, sidx4, didx4, bufs, gsems, ssems, n):
    gd = [None] * n
    sd = [None] * n
    for t in range(min(4, n)):
        gd[t] = pltpu.async_copy(y_hbm.at[sidx4.at[t]], bufs[t % 4],
                                 gsems[t % 4])
    for t in range(n):
        gd[t].wait()
        sd[t] = pltpu.async_copy(bufs[t % 4], acc.at[didx4.at[t]],
                                 ssems[t % 4], add=True)
        if t + 4 < n:
            sd[t].wait()
            gd[t + 4] = pltpu.async_copy(y_hbm.at[sidx4.at[t + 4]],
                                         bufs[t % 4], gsems[t % 4])
    for t in range(max(0, n - 4), n):
        sd[t].wait()


def _segsum_body(y_hbm, src2_hbm, dst2_hbm, out_hbm,
                 sidx4, didx4, buf0, buf1, buf2, buf3, zbuf, acc,
                 g0, g1, g2, g3, s0, s1, s2, s3):
    cid = lax.axis_index("c")
    sid = lax.axis_index("s")

    zv = jnp.zeros((16,), jnp.float32)

    def zb(i, carry):
        zbuf[i, pl.ds(0, 16)] = zv
        zbuf[i, pl.ds(16, 16)] = zv
        return carry

    lax.fori_loop(0, _ZR, zb, 0)

    def zacc(i, carry):
        pltpu.sync_copy(zbuf, acc.at[pl.ds(sid * _RPS + i * _ZR, _ZR)])
        return carry

    lax.fori_loop(0, _NZ, zacc, 0)

    @pl.when(sid == 0)
    def _():
        pltpu.sync_copy(zbuf.at[pl.ds(0, _REM)], acc.at[pl.ds(_REM0, _REM)])

    plsc.subcore_barrier()

    bufs = (buf0, buf1, buf2, buf3)
    gsems = (g0, g1, g2, g3)
    ssems = (s0, s1, s2, s3)
    extra = jnp.where(cid == 0, 2, 1)
    nb = 6 + jnp.where(sid < extra, 1, 0)
    core_base = cid * 98

    def sb_step(j, carry):
        row0 = (core_base + sid + _NS * j) * 32
        pltpu.sync_copy(src2_hbm.at[pl.ds(row0, 32)], sidx4)
        pltpu.sync_copy(dst2_hbm.at[pl.ds(row0, 32)], didx4)
        _edge_pipeline(y_hbm, acc, sidx4, didx4, bufs, gsems, ssems, 32)
        return carry

    lax.fori_loop(0, nb, sb_step, 0)

    @pl.when((cid == 1) & (sid == _NS - 1))
    def _():
        ntail = _CHUNKS - 195 * 32
        pltpu.sync_copy(src2_hbm.at[pl.ds(195 * 32, ntail)],
                        sidx4.at[pl.ds(0, ntail)])
        pltpu.sync_copy(dst2_hbm.at[pl.ds(195 * 32, ntail)],
                        didx4.at[pl.ds(0, ntail)])
        _edge_pipeline(y_hbm, acc, sidx4, didx4, bufs, gsems, ssems, ntail)

    plsc.subcore_barrier()

    def flush(i, carry):
        r0 = sid * _RPS + i * _ZR
        pltpu.sync_copy(acc.at[pl.ds(r0, _ZR)],
                        out_hbm.at[pl.ds(cid * N_NODES + r0, _ZR)])
        return carry

    lax.fori_loop(0, _NZ, flush, 0)

    @pl.when(sid == 0)
    def _():
        pltpu.sync_copy(acc.at[pl.ds(_REM0, _REM)],
                        out_hbm.at[pl.ds(cid * N_NODES + _REM0, _REM)])


def _deg_body(dst2_hbm, out_hbm, didx4, ones_buf, zbuf, acc, s0, s1, s2, s3):
    cid = lax.axis_index("c")
    sid = lax.axis_index("s")
    ssems = (s0, s1, s2, s3)

    zero = jnp.zeros((16,), jnp.float32)
    one = jnp.ones((16,), jnp.float32)

    def fb(i, carry):
        zbuf[i, pl.ds(0, 16)] = zero
        zbuf[i, pl.ds(16, 16)] = zero
        return carry

    lax.fori_loop(0, _ZR, fb, 0)

    def ob(i, carry):
        ones_buf[i, pl.ds(0, 16)] = one
        ones_buf[i, pl.ds(16, 16)] = one
        return carry

    lax.fori_loop(0, _CH, ob, 0)

    def zacc(i, carry):
        pltpu.sync_copy(zbuf, acc.at[pl.ds(sid * _RPS + i * _ZR, _ZR)])
        return carry

    lax.fori_loop(0, _NZ, zacc, 0)

    @pl.when(sid == 0)
    def _():
        pltpu.sync_copy(zbuf.at[pl.ds(0, _REM)], acc.at[pl.ds(_REM0, _REM)])

    plsc.subcore_barrier()

    extra = jnp.where(cid == 0, 2, 1)
    nb = 6 + jnp.where(sid < extra, 1, 0)
    core_base = cid * 98

    def scatter_run(n):
        sd = [None] * n
        for t in range(n):
            if t >= 8:
                sd[t - 8].wait()
            sd[t] = pltpu.async_copy(ones_buf, acc.at[didx4.at[t]],
                                     ssems[t % 4], add=True)
        for t in range(max(0, n - 8), n):
            sd[t].wait()

    def sb_step(j, carry):
        row0 = (core_base + sid + _NS * j) * 32
        pltpu.sync_copy(dst2_hbm.at[pl.ds(row0, 32)], didx4)
        scatter_run(32)
        return carry

    lax.fori_loop(0, nb, sb_step, 0)

    @pl.when((cid == 1) & (sid == _NS - 1))
    def _():
        ntail = _CHUNKS - 195 * 32
        pltpu.sync_copy(dst2_hbm.at[pl.ds(195 * 32, ntail)],
                        didx4.at[pl.ds(0, ntail)])
        scatter_run(ntail)

    plsc.subcore_barrier()

    def flush(i, carry):
        r0 = sid * _RPS + i * _ZR
        pltpu.sync_copy(acc.at[pl.ds(r0, _ZR)],
                        out_hbm.at[pl.ds(cid * N_NODES + r0, _ZR)])
        return carry

    lax.fori_loop(0, _NZ, flush, 0)

    @pl.when(sid == 0)
    def _():
        pltpu.sync_copy(acc.at[pl.ds(_REM0, _REM)],
                        out_hbm.at[pl.ds(cid * N_NODES + _REM0, _REM)])


_deg_call = pl.kernel(
    _deg_body,
    out_type=jax.ShapeDtypeStruct((_NC * N_NODES, DIM), jnp.float32),
    mesh=_sc_mesh,
    scratch_types=[
        pltpu.VMEM((32, _CH), jnp.int32),
        pltpu.VMEM((_CH, DIM), jnp.float32),
        pltpu.VMEM((_ZR, DIM), jnp.float32),
        pltpu.VMEM_SHARED((N_NODES, DIM), jnp.float32),
        pltpu.SemaphoreType.DMA,
        pltpu.SemaphoreType.DMA,
        pltpu.SemaphoreType.DMA,
        pltpu.SemaphoreType.DMA,
    ],
    compiler_params=pltpu.CompilerParams(use_tc_tiling_on_sc=False),
)


_segsum_call = pl.kernel(
    _segsum_body,
    out_type=jax.ShapeDtypeStruct((_NC * N_NODES, DIM), jnp.float32),
    mesh=_sc_mesh,
    scratch_types=[
        pltpu.VMEM((32, _CH), jnp.int32),
        pltpu.VMEM((32, _CH), jnp.int32),
        pltpu.VMEM((_CH, DIM), jnp.float32),
        pltpu.VMEM((_CH, DIM), jnp.float32),
        pltpu.VMEM((_CH, DIM), jnp.float32),
        pltpu.VMEM((_CH, DIM), jnp.float32),
        pltpu.VMEM((_ZR, DIM), jnp.float32),
        pltpu.VMEM_SHARED((N_NODES, DIM), jnp.float32),
        pltpu.SemaphoreType.DMA,
        pltpu.SemaphoreType.DMA,
        pltpu.SemaphoreType.DMA,
        pltpu.SemaphoreType.DMA,
        pltpu.SemaphoreType.DMA,
        pltpu.SemaphoreType.DMA,
        pltpu.SemaphoreType.DMA,
        pltpu.SemaphoreType.DMA,
    ],
    compiler_params=pltpu.CompilerParams(use_tc_tiling_on_sc=False),
)


_BLK = 5000
_NBLK = N_NODES // _BLK


def _matmul_body(x_ref, w_ref, o_ref):
    o_ref[...] = jnp.dot(x_ref[...], w_ref[...],
                         preferred_element_type=jnp.float32)


def _node_matmul(x, w):
    fin = x.shape[1]
    return pl.pallas_call(
        _matmul_body,
        grid=(_NBLK,),
        in_specs=[pl.BlockSpec((_BLK, fin), lambda i: (i, 0)),
                  pl.BlockSpec((fin, DIM), lambda i: (0, 0))],
        out_specs=pl.BlockSpec((_BLK, DIM), lambda i: (i, 0)),
        out_shape=jax.ShapeDtypeStruct((N_NODES, DIM), jnp.float32),
    )(x, w)


def _stats_accum(m, sum_ref, ssq_ref):
    ones = jnp.ones((8, m.shape[0]), jnp.float32)
    ps = jnp.dot(ones, m, preferred_element_type=jnp.float32)
    pq = jnp.dot(ones, m * m, preferred_element_type=jnp.float32)

    @pl.when(pl.program_id(0) == 0)
    def _():
        sum_ref[...] = ps
        ssq_ref[...] = pq

    @pl.when(pl.program_id(0) != 0)
    def _():
        sum_ref[...] += ps
        ssq_ref[...] += pq


def _gin0_body(y_ref, s0_ref, s1_ref, b1_ref, w2_ref, b2_ref,
               m_ref, sum_ref, ssq_ref):
    u = jnp.maximum(y_ref[...] + s0_ref[...] + s1_ref[...] + b1_ref[...], 0.0)
    m = jnp.maximum(
        jnp.dot(u, w2_ref[...], preferred_element_type=jnp.float32)
        + b2_ref[...], 0.0)
    m_ref[...] = m
    _stats_accum(m, sum_ref, ssq_ref)


def _gin_body(m_ref, s0_ref, s1_ref, d0_ref, d1_ref, sum_ref, ssq_ref,
              g_ref, b_ref, w1_ref, b1_ref, w2_ref, b2_ref,
              mo_ref, sumo_ref, ssqo_ref):
    a, c = _bn_scale(sum_ref[...], ssq_ref[...], g_ref[...], b_ref[...])
    t = (a * (m_ref[...] + s0_ref[...] + s1_ref[...])
         + c * (1.0 + d0_ref[...] + d1_ref[...]))
    u = jnp.maximum(
        jnp.dot(t, w1_ref[...], preferred_element_type=jnp.float32)
        + b1_ref[...], 0.0)
    m = jnp.maximum(
        jnp.dot(u, w2_ref[...], preferred_element_type=jnp.float32)
        + b2_ref[...], 0.0)
    mo_ref[...] = m
    _stats_accum(m, sumo_ref, ssqo_ref)


_nodes_spec = pl.BlockSpec((_BLK, DIM), lambda i: (i, 0))
_part0_spec = pl.BlockSpec((_BLK, DIM), lambda i: (i, 0))
_part1_spec = pl.BlockSpec((_BLK, DIM), lambda i: (i + _NBLK, 0))
_w_spec = pl.BlockSpec((DIM, DIM), lambda i: (0, 0))
_b_spec = pl.BlockSpec((1, DIM), lambda i: (0, 0))
_stat_spec = pl.BlockSpec((8, DIM), lambda i: (0, 0))
_mlp_out_shapes = (
    jax.ShapeDtypeStruct((N_NODES, DIM), jnp.float32),
    jax.ShapeDtypeStruct((8, DIM), jnp.float32),
    jax.ShapeDtypeStruct((8, DIM), jnp.float32),
)


def _gin_mlp0(y, s, b1, w2, b2):
    return pl.pallas_call(
        _gin0_body,
        grid=(_NBLK,),
        in_specs=[_nodes_spec, _part0_spec, _part1_spec,
                  _b_spec, _w_spec, _b_spec],
        out_specs=(_nodes_spec, _stat_spec, _stat_spec),
        out_shape=_mlp_out_shapes,
    )(y, s, s, b1, w2, b2)


def _gin_mlp(m, s, d, s8, q8, g, b, w1, b1, w2, b2):
    return pl.pallas_call(
        _gin_body,
        grid=(_NBLK,),
        in_specs=[_nodes_spec, _part0_spec, _part1_spec,
                  _part0_spec, _part1_spec,
                  _stat_spec, _stat_spec, _b_spec, _b_spec,
                  _w_spec, _b_spec, _w_spec, _b_spec],
        out_specs=(_nodes_spec, _stat_spec, _stat_spec),
        out_shape=_mlp_out_shapes,
    )(m, s, s, d, d, s8, q8, g, b, w1, b1, w2, b2)


def _bn_scale(sum_v, ssq_v, g_v, b_v):
    s = jnp.sum(sum_v, axis=0, keepdims=True)
    q = jnp.sum(ssq_v, axis=0, keepdims=True)
    mu = s * (1.0 / N_NODES)
    var = q * (1.0 / N_NODES) - mu * mu
    a = g_v * lax.rsqrt(var + 1e-5)
    c = b_v - mu * a
    return a, c


def _bn_pool_body(m_ref, sum_ref, ssq_ref, g_ref, b_ref, bt_ref, p_ref):
    a, c = _bn_scale(sum_ref[...], ssq_ref[...], g_ref[...], b_ref[...])
    h = m_ref[...] * a + c
    brow = bt_ref[...].reshape(1, _BLK)
    oh = (lax.broadcasted_iota(jnp.int32, (N_GRAPHS, _BLK), 0)
          == brow).astype(jnp.float32)
    pp = jnp.dot(oh, h, preferred_element_type=jnp.float32)

    @pl.when(pl.program_id(0) == 0)
    def _():
        p_ref[...] = pp

    @pl.when(pl.program_id(0) != 0)
    def _():
        p_ref[...] += pp


def _bn_pool(m, s8, q8, g, b, batch3):
    return pl.pallas_call(
        _bn_pool_body,
        grid=(_NBLK,),
        in_specs=[_nodes_spec, _stat_spec, _stat_spec, _b_spec, _b_spec,
                  pl.BlockSpec((1, 1, _BLK), lambda i: (i, 0, 0))],
        out_specs=pl.BlockSpec((N_GRAPHS, DIM), lambda i: (0, 0)),
        out_shape=jax.ShapeDtypeStruct((N_GRAPHS, DIM), jnp.float32),
    )(m, s8, q8, g, b, batch3)


_HG = 8


def _heads_body(t2_ref, t1_ref, w2s_ref, a2t_ref, b2c_ref, w1s_ref, b1c_ref,
                c2_ref, c1_ref):
    t2 = t2_ref[...].reshape(_HG, 1000)
    t1 = t1_ref[...]
    for g in range(_HG):
        t2g = t2[g:g + 1, :]
        oh = (lax.broadcasted_iota(jnp.int32, (32, 1000), 0)
              == t2g).astype(jnp.float32)
        c2 = jnp.zeros((32, 121), jnp.float32)
        for k in range(8):
            bk = lax.dot_general(w2s_ref[k], oh, (((1,), (1,)), ((), ())),
                                 preferred_element_type=jnp.float32)
            c2 = c2 + jnp.dot(bk, a2t_ref[k],
                              preferred_element_type=jnp.float32)
        c2_ref[g] = c2 + b2c_ref[...]

        t1g = t1[g]
        c1 = jnp.zeros((32, 17), jnp.float32)
        for k in range(8):
            c1 = c1 + jnp.dot(w1s_ref[k], t1g[:, k:k + 17],
                              preferred_element_type=jnp.float32)
        c1_ref[g] = c1 + b1c_ref[...]


def _conv_heads(t2r, t1, w2s, a2t, b2c, w1s, b1c):
    return pl.pallas_call(
        _heads_body,
        grid=(N_GRAPHS // _HG,),
        in_specs=[
            pl.BlockSpec((_HG, 1, 1000), lambda i: (i, 0, 0)),
            pl.BlockSpec((_HG, 20, 24), lambda i: (i, 0, 0)),
            pl.BlockSpec((8, 32, 1000), lambda i: (0, 0, 0)),
            pl.BlockSpec((8, 32, 121), lambda i: (0, 0, 0)),
            pl.BlockSpec((32, 1), lambda i: (0, 0)),
            pl.BlockSpec((8, 32, 20), lambda i: (0, 0, 0)),
            pl.BlockSpec((32, 1), lambda i: (0, 0)),
        ],
        out_specs=(pl.BlockSpec((_HG, 32, 121), lambda i: (i, 0, 0)),
                   pl.BlockSpec((_HG, 32, 17), lambda i: (i, 0, 0))),
        out_shape=(jax.ShapeDtypeStruct((N_GRAPHS, 32, 121), jnp.float32),
                   jax.ShapeDtypeStruct((N_GRAPHS, 32, 17), jnp.float32)),
    )(t2r, t1, w2s, a2t, b2c, w1s, b1c)


def _final_body(p_ref, c1f_ref, c2f_ref, wxd_ref, bxd_ref, w1x_ref, b1x_ref,
                w2x_ref, b2x_ref, wa_ref, wb_ref, wc_ref, bf1_ref,
                wf2_ref, bf2_ref, wo_ref, bo_ref, o_ref):
    f32 = jnp.float32
    xd = jnp.maximum(
        jnp.dot(p_ref[...], wxd_ref[...], preferred_element_type=f32)
        + bxd_ref[...], 0.0)
    xt1 = jnp.dot(c1f_ref[...], w1x_ref[...],
                  preferred_element_type=f32) + b1x_ref[...]
    xt2 = jnp.dot(c2f_ref[...], w2x_ref[...],
                  preferred_element_type=f32) + b2x_ref[...]
    z = jnp.maximum(
        jnp.dot(xd, wa_ref[...], preferred_element_type=f32)
        + jnp.dot(xt1, wb_ref[...], preferred_element_type=f32)
        + jnp.dot(xt2, wc_ref[...], preferred_element_type=f32)
        + bf1_ref[...], 0.0)
    z2 = jnp.maximum(
        jnp.dot(z, wf2_ref[...], preferred_element_type=f32)
        + bf2_ref[...], 0.0)
    o_ref[...] = jnp.dot(z2, wo_ref[...],
                         preferred_element_type=f32) + bo_ref[...]


def _final_mlp(pooled, c1f, c2f, wxd, bxd, w1x, b1x, w2x, b2x,
               wa, wb, wc, bf1, wf2, bf2, wo, bo):
    return pl.pallas_call(
        _final_body,
        out_shape=jax.ShapeDtypeStruct((N_GRAPHS, 1), jnp.float32),
    )(pooled, c1f, c2f, wxd, bxd, w1x, b1x, w2x, b2x,
      wa, wb, wc, bf1, wf2, bf2, wo, bo)


def kernel(x, edge_index, batch, target1, target2, params):
    p = params
    src2 = edge_index[0].reshape(_CHUNKS, _CH)
    dst2 = edge_index[1].reshape(_CHUNKS, _CH)

    def row(v):
        return v.reshape(1, DIM)

    batch3 = batch.astype(jnp.int32).reshape(_NBLK, 1, _BLK)
    deg = _deg_call(dst2)
    y = _node_matmul(x, p['gin0_w1'])
    s = _segsum_call(y, src2, dst2)
    m, s8, q8 = _gin_mlp0(y, s, row(p['gin0_b1']),
                          p['gin0_w2'], row(p['gin0_b2']))
    for i in range(1, 5):
        s = _segsum_call(m, src2, dst2)
        m, s8, q8 = _gin_mlp(m, s, deg, s8, q8,
                             row(p['bn%d_g' % (i - 1)]),
                             row(p['bn%d_b' % (i - 1)]),
                             p['gin%d_w1' % i], row(p['gin%d_b1' % i]),
                             p['gin%d_w2' % i], row(p['gin%d_b2' % i]))
    pooled = _bn_pool(m, s8, q8, row(p['bn4_g']), row(p['bn4_b']), batch3)

    t2r = target2.astype(jnp.int32).reshape(N_GRAPHS, 1, 1000)
    epad = jnp.concatenate(
        [p['emb'], jnp.zeros((6, 128), jnp.float32)], axis=0)
    a2t = jnp.stack([epad[:, k:k + 121] for k in range(8)])
    w2s = jnp.transpose(p['cxt2_w'], (2, 0, 1))
    w1s = jnp.transpose(p['cxt1_w'], (2, 0, 1))
    c2, c1 = _conv_heads(t2r, target1, w2s, a2t,
                         p['cxt2_b'].reshape(32, 1), w1s,
                         p['cxt1_b'].reshape(32, 1))
    c2f = c2.reshape(N_GRAPHS, 32 * 121)
    c1f = c1.reshape(N_GRAPHS, 32 * 17)

    fw = p['fc1_w']
    out = _final_mlp(
        pooled, c1f, c2f,
        p['fc1_xd_w'], p['fc1_xd_b'].reshape(1, -1),
        p['fc1_xt_w'], p['fc1_xt_b'].reshape(1, -1),
        p['fc2_xt_w'], p['fc2_xt_b'].reshape(1, -1),
        fw[0:128], fw[128:256], fw[256:384], p['fc1_b'].reshape(1, -1),
        p['fc2_w'], p['fc2_b'].reshape(1, -1),
        p['out_w'], p['out_b'].reshape(1, -1))
    return out


# speedup vs baseline: 15.4237x; 1.0373x over previous
import jax
import jax.numpy as jnp
from jax import lax
from jax.experimental import pallas as pl
from jax.experimental.pallas import tpu as pltpu
from jax.experimental.pallas import tpu_sc as plsc

N_GRAPHS = 128
N_NODES = 50000
DIM = 32
N_EDGES = 800000

_NC, _NS = 2, 16
_CH = 128
_CHUNKS = N_EDGES // _CH
_CPC = _CHUNKS // _NC
_RPS = 3120
_ZR = 120
_NZ = _RPS // _ZR
_REM0 = _NS * _RPS
_REM = N_NODES - _REM0

_sc_mesh = plsc.VectorSubcoreMesh(
    core_axis_name="c", subcore_axis_name="s", num_cores=_NC, num_subcores=_NS)


def _edge_pipeline(y_hbm, acc, sidx4, didx4, bufs, gsems, ssems, n):
    gd = [None] * n
    sd = [None] * n
    for t in range(min(4, n)):
        gd[t] = pltpu.async_copy(y_hbm.at[sidx4.at[t]], bufs[t % 4],
                                 gsems[t % 4])
    for t in range(n):
        gd[t].wait()
        sd[t] = pltpu.async_copy(bufs[t % 4], acc.at[didx4.at[t]],
                                 ssems[t % 4], add=True)
        if t + 4 < n:
            sd[t].wait()
            gd[t + 4] = pltpu.async_copy(y_hbm.at[sidx4.at[t + 4]],
                                         bufs[t % 4], gsems[t % 4])
    for t in range(max(0, n - 4), n):
        sd[t].wait()


def _segsum_body(y_hbm, src2_hbm, dst2_hbm, out_hbm,
                 sidx4, didx4, buf0, buf1, buf2, buf3, zbuf, acc,
                 g0, g1, g2, g3, s0, s1, s2, s3):
    cid = lax.axis_index("c")
    sid = lax.axis_index("s")

    zv = jnp.zeros((16,), jnp.float32)

    def zb(i, carry):
        zbuf[i, pl.ds(0, 16)] = zv
        zbuf[i, pl.ds(16, 16)] = zv
        return carry

    lax.fori_loop(0, _ZR, zb, 0)

    zsems = (s0, s1, s2, s3)
    zd = [None] * _NZ
    for i in range(_NZ):
        if i >= 4:
            zd[i - 4].wait()
        zd[i] = pltpu.async_copy(
            zbuf, acc.at[pl.ds(sid * _RPS + i * _ZR, _ZR)], zsems[i % 4])
    for i in range(_NZ - 4, _NZ):
        zd[i].wait()

    @pl.when(sid == 0)
    def _():
        pltpu.sync_copy(zbuf.at[pl.ds(0, _REM)], acc.at[pl.ds(_REM0, _REM)])

    plsc.subcore_barrier()

    bufs = (buf0, buf1, buf2, buf3)
    gsems = (g0, g1, g2, g3)
    ssems = (s0, s1, s2, s3)
    extra = jnp.where(cid == 0, 2, 1)
    nb = 6 + jnp.where(sid < extra, 1, 0)
    core_base = cid * 98

    def sb_step(j, carry):
        row0 = (core_base + sid + _NS * j) * 32
        pltpu.sync_copy(src2_hbm.at[pl.ds(row0, 32)], sidx4)
        pltpu.sync_copy(dst2_hbm.at[pl.ds(row0, 32)], didx4)
        _edge_pipeline(y_hbm, acc, sidx4, didx4, bufs, gsems, ssems, 32)
        return carry

    lax.fori_loop(0, nb, sb_step, 0)

    @pl.when((cid == 1) & (sid == _NS - 1))
    def _():
        ntail = _CHUNKS - 195 * 32
        pltpu.sync_copy(src2_hbm.at[pl.ds(195 * 32, ntail)],
                        sidx4.at[pl.ds(0, ntail)])
        pltpu.sync_copy(dst2_hbm.at[pl.ds(195 * 32, ntail)],
                        didx4.at[pl.ds(0, ntail)])
        _edge_pipeline(y_hbm, acc, sidx4, didx4, bufs, gsems, ssems, ntail)

    plsc.subcore_barrier()

    fd = [None] * _NZ
    for i in range(_NZ):
        if i >= 4:
            fd[i - 4].wait()
        r0 = sid * _RPS + i * _ZR
        fd[i] = pltpu.async_copy(
            acc.at[pl.ds(r0, _ZR)],
            out_hbm.at[pl.ds(cid * N_NODES + r0, _ZR)], ssems[i % 4])
    for i in range(_NZ - 4, _NZ):
        fd[i].wait()

    @pl.when(sid == 0)
    def _():
        pltpu.sync_copy(acc.at[pl.ds(_REM0, _REM)],
                        out_hbm.at[pl.ds(cid * N_NODES + _REM0, _REM)])


def _deg_body(dst2_hbm, out_hbm, didx4, ones_buf, zbuf, acc, s0, s1, s2, s3):
    cid = lax.axis_index("c")
    sid = lax.axis_index("s")
    ssems = (s0, s1, s2, s3)

    zero = jnp.zeros((16,), jnp.float32)
    one = jnp.ones((16,), jnp.float32)

    def fb(i, carry):
        zbuf[i, pl.ds(0, 16)] = zero
        zbuf[i, pl.ds(16, 16)] = zero
        return carry

    lax.fori_loop(0, _ZR, fb, 0)

    def ob(i, carry):
        ones_buf[i, pl.ds(0, 16)] = one
        ones_buf[i, pl.ds(16, 16)] = one
        return carry

    lax.fori_loop(0, _CH, ob, 0)

    zd = [None] * _NZ
    for i in range(_NZ):
        if i >= 4:
            zd[i - 4].wait()
        zd[i] = pltpu.async_copy(
            zbuf, acc.at[pl.ds(sid * _RPS + i * _ZR, _ZR)], ssems[i % 4])
    for i in range(_NZ - 4, _NZ):
        zd[i].wait()

    @pl.when(sid == 0)
    def _():
        pltpu.sync_copy(zbuf.at[pl.ds(0, _REM)], acc.at[pl.ds(_REM0, _REM)])

    plsc.subcore_barrier()

    extra = jnp.where(cid == 0, 2, 1)
    nb = 6 + jnp.where(sid < extra, 1, 0)
    core_base = cid * 98

    def scatter_run(n):
        sd = [None] * n
        for t in range(n):
            if t >= 8:
                sd[t - 8].wait()
            sd[t] = pltpu.async_copy(ones_buf, acc.at[didx4.at[t]],
                                     ssems[t % 4], add=True)
        for t in range(max(0, n - 8), n):
            sd[t].wait()

    def sb_step(j, carry):
        row0 = (core_base + sid + _NS * j) * 32
        pltpu.sync_copy(dst2_hbm.at[pl.ds(row0, 32)], didx4)
        scatter_run(32)
        return carry

    lax.fori_loop(0, nb, sb_step, 0)

    @pl.when((cid == 1) & (sid == _NS - 1))
    def _():
        ntail = _CHUNKS - 195 * 32
        pltpu.sync_copy(dst2_hbm.at[pl.ds(195 * 32, ntail)],
                        didx4.at[pl.ds(0, ntail)])
        scatter_run(ntail)

    plsc.subcore_barrier()

    fd = [None] * _NZ
    for i in range(_NZ):
        if i >= 4:
            fd[i - 4].wait()
        r0 = sid * _RPS + i * _ZR
        fd[i] = pltpu.async_copy(
            acc.at[pl.ds(r0, _ZR)],
            out_hbm.at[pl.ds(cid * N_NODES + r0, _ZR)], ssems[i % 4])
    for i in range(_NZ - 4, _NZ):
        fd[i].wait()

    @pl.when(sid == 0)
    def _():
        pltpu.sync_copy(acc.at[pl.ds(_REM0, _REM)],
                        out_hbm.at[pl.ds(cid * N_NODES + _REM0, _REM)])


_deg_call = pl.kernel(
    _deg_body,
    out_type=jax.ShapeDtypeStruct((_NC * N_NODES, DIM), jnp.float32),
    mesh=_sc_mesh,
    scratch_types=[
        pltpu.VMEM((32, _CH), jnp.int32),
        pltpu.VMEM((_CH, DIM), jnp.float32),
        pltpu.VMEM((_ZR, DIM), jnp.float32),
        pltpu.VMEM_SHARED((N_NODES, DIM), jnp.float32),
        pltpu.SemaphoreType.DMA,
        pltpu.SemaphoreType.DMA,
        pltpu.SemaphoreType.DMA,
        pltpu.SemaphoreType.DMA,
    ],
    compiler_params=pltpu.CompilerParams(use_tc_tiling_on_sc=False),
)


_segsum_call = pl.kernel(
    _segsum_body,
    out_type=jax.ShapeDtypeStruct((_NC * N_NODES, DIM), jnp.float32),
    mesh=_sc_mesh,
    scratch_types=[
        pltpu.VMEM((32, _CH), jnp.int32),
        pltpu.VMEM((32, _CH), jnp.int32),
        pltpu.VMEM((_CH, DIM), jnp.float32),
        pltpu.VMEM((_CH, DIM), jnp.float32),
        pltpu.VMEM((_CH, DIM), jnp.float32),
        pltpu.VMEM((_CH, DIM), jnp.float32),
        pltpu.VMEM((_ZR, DIM), jnp.float32),
        pltpu.VMEM_SHARED((N_NODES, DIM), jnp.float32),
        pltpu.SemaphoreType.DMA,
        pltpu.SemaphoreType.DMA,
        pltpu.SemaphoreType.DMA,
        pltpu.SemaphoreType.DMA,
        pltpu.SemaphoreType.DMA,
        pltpu.SemaphoreType.DMA,
        pltpu.SemaphoreType.DMA,
        pltpu.SemaphoreType.DMA,
    ],
    compiler_params=pltpu.CompilerParams(use_tc_tiling_on_sc=False),
)


_BLK = 5000
_NBLK = N_NODES // _BLK


def _matmul_body(x_ref, w_ref, o_ref):
    o_ref[...] = jnp.dot(x_ref[...], w_ref[...],
                         preferred_element_type=jnp.float32)


def _node_matmul(x, w):
    fin = x.shape[1]
    return pl.pallas_call(
        _matmul_body,
        grid=(_NBLK,),
        in_specs=[pl.BlockSpec((_BLK, fin), lambda i: (i, 0)),
                  pl.BlockSpec((fin, DIM), lambda i: (0, 0))],
        out_specs=pl.BlockSpec((_BLK, DIM), lambda i: (i, 0)),
        out_shape=jax.ShapeDtypeStruct((N_NODES, DIM), jnp.float32),
    )(x, w)


def _stats_accum(m, sum_ref, ssq_ref):
    ones = jnp.ones((8, m.shape[0]), jnp.float32)
    ps = jnp.dot(ones, m, preferred_element_type=jnp.float32)
    pq = jnp.dot(ones, m * m, preferred_element_type=jnp.float32)

    @pl.when(pl.program_id(0) == 0)
    def _():
        sum_ref[...] = ps
        ssq_ref[...] = pq

    @pl.when(pl.program_id(0) != 0)
    def _():
        sum_ref[...] += ps
        ssq_ref[...] += pq


def _gin0_body(y_ref, s0_ref, s1_ref, b1_ref, w2_ref, b2_ref,
               m_ref, sum_ref, ssq_ref):
    u = jnp.maximum(y_ref[...] + s0_ref[...] + s1_ref[...] + b1_ref[...], 0.0)
    m = jnp.maximum(
        jnp.dot(u, w2_ref[...], preferred_element_type=jnp.float32)
        + b2_ref[...], 0.0)
    m_ref[...] = m
    _stats_accum(m, sum_ref, ssq_ref)


def _gin_body(m_ref, s0_ref, s1_ref, d0_ref, d1_ref, sum_ref, ssq_ref,
              g_ref, b_ref, w1_ref, b1_ref, w2_ref, b2_ref,
              mo_ref, sumo_ref, ssqo_ref):
    a, c = _bn_scale(sum_ref[...], ssq_ref[...], g_ref[...], b_ref[...])
    t = (a * (m_ref[...] + s0_ref[...] + s1_ref[...])
         + c * (1.0 + d0_ref[...] + d1_ref[...]))
    u = jnp.maximum(
        jnp.dot(t, w1_ref[...], preferred_element_type=jnp.float32)
        + b1_ref[...], 0.0)
    m = jnp.maximum(
        jnp.dot(u, w2_ref[...], preferred_element_type=jnp.float32)
        + b2_ref[...], 0.0)
    mo_ref[...] = m
    _stats_accum(m, sumo_ref, ssqo_ref)


_nodes_spec = pl.BlockSpec((_BLK, DIM), lambda i: (i, 0))
_part0_spec = pl.BlockSpec((_BLK, DIM), lambda i: (i, 0))
_part1_spec = pl.BlockSpec((_BLK, DIM), lambda i: (i + _NBLK, 0))
_w_spec = pl.BlockSpec((DIM, DIM), lambda i: (0, 0))
_b_spec = pl.BlockSpec((1, DIM), lambda i: (0, 0))
_stat_spec = pl.BlockSpec((8, DIM), lambda i: (0, 0))
_mlp_out_shapes = (
    jax.ShapeDtypeStruct((N_NODES, DIM), jnp.float32),
    jax.ShapeDtypeStruct((8, DIM), jnp.float32),
    jax.ShapeDtypeStruct((8, DIM), jnp.float32),
)


def _gin_mlp0(y, s, b1, w2, b2):
    return pl.pallas_call(
        _gin0_body,
        grid=(_NBLK,),
        in_specs=[_nodes_spec, _part0_spec, _part1_spec,
                  _b_spec, _w_spec, _b_spec],
        out_specs=(_nodes_spec, _stat_spec, _stat_spec),
        out_shape=_mlp_out_shapes,
    )(y, s, s, b1, w2, b2)


def _gin_mlp(m, s, d, s8, q8, g, b, w1, b1, w2, b2):
    return pl.pallas_call(
        _gin_body,
        grid=(_NBLK,),
        in_specs=[_nodes_spec, _part0_spec, _part1_spec,
                  _part0_spec, _part1_spec,
                  _stat_spec, _stat_spec, _b_spec, _b_spec,
                  _w_spec, _b_spec, _w_spec, _b_spec],
        out_specs=(_nodes_spec, _stat_spec, _stat_spec),
        out_shape=_mlp_out_shapes,
    )(m, s, s, d, d, s8, q8, g, b, w1, b1, w2, b2)


def _bn_scale(sum_v, ssq_v, g_v, b_v):
    s = jnp.sum(sum_v, axis=0, keepdims=True)
    q = jnp.sum(ssq_v, axis=0, keepdims=True)
    mu = s * (1.0 / N_NODES)
    var = q * (1.0 / N_NODES) - mu * mu
    a = g_v * lax.rsqrt(var + 1e-5)
    c = b_v - mu * a
    return a, c


def _bn_pool_body(m_ref, sum_ref, ssq_ref, g_ref, b_ref, bt_ref, p_ref):
    a, c = _bn_scale(sum_ref[...], ssq_ref[...], g_ref[...], b_ref[...])
    h = m_ref[...] * a + c
    brow = bt_ref[...].reshape(1, _BLK)
    oh = (lax.broadcasted_iota(jnp.int32, (N_GRAPHS, _BLK), 0)
          == brow).astype(jnp.float32)
    pp = jnp.dot(oh, h, preferred_element_type=jnp.float32)

    @pl.when(pl.program_id(0) == 0)
    def _():
        p_ref[...] = pp

    @pl.when(pl.program_id(0) != 0)
    def _():
        p_ref[...] += pp


def _bn_pool(m, s8, q8, g, b, batch3):
    return pl.pallas_call(
        _bn_pool_body,
        grid=(_NBLK,),
        in_specs=[_nodes_spec, _stat_spec, _stat_spec, _b_spec, _b_spec,
                  pl.BlockSpec((1, 1, _BLK), lambda i: (i, 0, 0))],
        out_specs=pl.BlockSpec((N_GRAPHS, DIM), lambda i: (0, 0)),
        out_shape=jax.ShapeDtypeStruct((N_GRAPHS, DIM), jnp.float32),
    )(m, s8, q8, g, b, batch3)


_HG = 8


def _heads_body(t2_ref, t1_ref, w2s_ref, a2t_ref, b2c_ref, w1s_ref, b1c_ref,
                c2_ref, c1_ref):
    t2 = t2_ref[...].reshape(_HG, 1000)
    t1 = t1_ref[...]
    for g in range(_HG):
        t2g = t2[g:g + 1, :]
        oh = (lax.broadcasted_iota(jnp.int32, (32, 1000), 0)
              == t2g).astype(jnp.float32)
        c2 = jnp.zeros((32, 121), jnp.float32)
        for k in range(8):
            bk = lax.dot_general(w2s_ref[k], oh, (((1,), (1,)), ((), ())),
                                 preferred_element_type=jnp.float32)
            c2 = c2 + jnp.dot(bk, a2t_ref[k],
                              preferred_element_type=jnp.float32)
        c2_ref[g] = c2 + b2c_ref[...]

        t1g = t1[g]
        c1 = jnp.zeros((32, 17), jnp.float32)
        for k in range(8):
            c1 = c1 + jnp.dot(w1s_ref[k], t1g[:, k:k + 17],
                              preferred_element_type=jnp.float32)
        c1_ref[g] = c1 + b1c_ref[...]


def _conv_heads(t2r, t1, w2s, a2t, b2c, w1s, b1c):
    return pl.pallas_call(
        _heads_body,
        grid=(N_GRAPHS // _HG,),
        in_specs=[
            pl.BlockSpec((_HG, 1, 1000), lambda i: (i, 0, 0)),
            pl.BlockSpec((_HG, 20, 24), lambda i: (i, 0, 0)),
            pl.BlockSpec((8, 32, 1000), lambda i: (0, 0, 0)),
            pl.BlockSpec((8, 32, 121), lambda i: (0, 0, 0)),
            pl.BlockSpec((32, 1), lambda i: (0, 0)),
            pl.BlockSpec((8, 32, 20), lambda i: (0, 0, 0)),
            pl.BlockSpec((32, 1), lambda i: (0, 0)),
        ],
        out_specs=(pl.BlockSpec((_HG, 32, 121), lambda i: (i, 0, 0)),
                   pl.BlockSpec((_HG, 32, 17), lambda i: (i, 0, 0))),
        out_shape=(jax.ShapeDtypeStruct((N_GRAPHS, 32, 121), jnp.float32),
                   jax.ShapeDtypeStruct((N_GRAPHS, 32, 17), jnp.float32)),
    )(t2r, t1, w2s, a2t, b2c, w1s, b1c)


def _final_body(p_ref, c1f_ref, c2f_ref, wxd_ref, bxd_ref, w1x_ref, b1x_ref,
                w2x_ref, b2x_ref, wa_ref, wb_ref, wc_ref, bf1_ref,
                wf2_ref, bf2_ref, wo_ref, bo_ref, o_ref):
    f32 = jnp.float32
    xd = jnp.maximum(
        jnp.dot(p_ref[...], wxd_ref[...], preferred_element_type=f32)
        + bxd_ref[...], 0.0)
    xt1 = jnp.dot(c1f_ref[...], w1x_ref[...],
                  preferred_element_type=f32) + b1x_ref[...]
    xt2 = jnp.dot(c2f_ref[...], w2x_ref[...],
                  preferred_element_type=f32) + b2x_ref[...]
    z = jnp.maximum(
        jnp.dot(xd, wa_ref[...], preferred_element_type=f32)
        + jnp.dot(xt1, wb_ref[...], preferred_element_type=f32)
        + jnp.dot(xt2, wc_ref[...], preferred_element_type=f32)
        + bf1_ref[...], 0.0)
    z2 = jnp.maximum(
        jnp.dot(z, wf2_ref[...], preferred_element_type=f32)
        + bf2_ref[...], 0.0)
    o_ref[...] = jnp.dot(z2, wo_ref[...],
                         preferred_element_type=f32) + bo_ref[...]


def _final_mlp(pooled, c1f, c2f, wxd, bxd, w1x, b1x, w2x, b2x,
               wa, wb, wc, bf1, wf2, bf2, wo, bo):
    return pl.pallas_call(
        _final_body,
        out_shape=jax.ShapeDtypeStruct((N_GRAPHS, 1), jnp.float32),
    )(pooled, c1f, c2f, wxd, bxd, w1x, b1x, w2x, b2x,
      wa, wb, wc, bf1, wf2, bf2, wo, bo)


def kernel(x, edge_index, batch, target1, target2, params):
    p = params
    src2 = edge_index[0].reshape(_CHUNKS, _CH)
    dst2 = edge_index[1].reshape(_CHUNKS, _CH)

    def row(v):
        return v.reshape(1, DIM)

    batch3 = batch.astype(jnp.int32).reshape(_NBLK, 1, _BLK)
    deg = _deg_call(dst2)
    y = _node_matmul(x, p['gin0_w1'])
    s = _segsum_call(y, src2, dst2)
    m, s8, q8 = _gin_mlp0(y, s, row(p['gin0_b1']),
                          p['gin0_w2'], row(p['gin0_b2']))
    for i in range(1, 5):
        s = _segsum_call(m, src2, dst2)
        m, s8, q8 = _gin_mlp(m, s, deg, s8, q8,
                             row(p['bn%d_g' % (i - 1)]),
                             row(p['bn%d_b' % (i - 1)]),
                             p['gin%d_w1' % i], row(p['gin%d_b1' % i]),
                             p['gin%d_w2' % i], row(p['gin%d_b2' % i]))
    pooled = _bn_pool(m, s8, q8, row(p['bn4_g']), row(p['bn4_b']), batch3)

    t2r = target2.astype(jnp.int32).reshape(N_GRAPHS, 1, 1000)
    epad = jnp.concatenate(
        [p['emb'], jnp.zeros((6, 128), jnp.float32)], axis=0)
    a2t = jnp.stack([epad[:, k:k + 121] for k in range(8)])
    w2s = jnp.transpose(p['cxt2_w'], (2, 0, 1))
    w1s = jnp.transpose(p['cxt1_w'], (2, 0, 1))
    c2, c1 = _conv_heads(t2r, target1, w2s, a2t,
                         p['cxt2_b'].reshape(32, 1), w1s,
                         p['cxt1_b'].reshape(32, 1))
    c2f = c2.reshape(N_GRAPHS, 32 * 121)
    c1f = c1.reshape(N_GRAPHS, 32 * 17)

    fw = p['fc1_w']
    out = _final_mlp(
        pooled, c1f, c2f,
        p['fc1_xd_w'], p['fc1_xd_b'].reshape(1, -1),
        p['fc1_xt_w'], p['fc1_xt_b'].reshape(1, -1),
        p['fc2_xt_w'], p['fc2_xt_b'].reshape(1, -1),
        fw[0:128], fw[128:256], fw[256:384], p['fc1_b'].reshape(1, -1),
        p['fc2_w'], p['fc2_b'].reshape(1, -1),
        p['out_w'], p['out_b'].reshape(1, -1))
    return out


# speedup vs baseline: 15.4425x; 1.0012x over previous
import jax
import jax.numpy as jnp
from jax import lax
from jax.experimental import pallas as pl
from jax.experimental.pallas import tpu as pltpu
from jax.experimental.pallas import tpu_sc as plsc

N_GRAPHS = 128
N_NODES = 50000
DIM = 32
N_EDGES = 800000

_NC, _NS = 2, 16
_CH = 128
_CHUNKS = N_EDGES // _CH
_CPC = _CHUNKS // _NC
_RPS = 3120
_ZR = 120
_NZ = _RPS // _ZR
_REM0 = _NS * _RPS
_REM = N_NODES - _REM0

_sc_mesh = plsc.VectorSubcoreMesh(
    core_axis_name="c", subcore_axis_name="s", num_cores=_NC, num_subcores=_NS)


def _edge_pipeline(y_hbm, acc, sidx4, didx4, bufs, gsems, ssems, n):
    gd = [None] * n
    sd = [None] * n
    for t in range(min(4, n)):
        gd[t] = pltpu.async_copy(y_hbm.at[sidx4.at[t]], bufs[t % 4],
                                 gsems[t % 4])
    for t in range(n):
        gd[t].wait()
        sd[t] = pltpu.async_copy(bufs[t % 4], acc.at[didx4.at[t]],
                                 ssems[t % 4], add=True)
        if t + 4 < n:
            sd[t].wait()
            gd[t + 4] = pltpu.async_copy(y_hbm.at[sidx4.at[t + 4]],
                                         bufs[t % 4], gsems[t % 4])
    for t in range(max(0, n - 4), n):
        sd[t].wait()


def _segsum_body(y_hbm, src2_hbm, dst2_hbm, out_hbm,
                 sidx4, didx4, buf0, buf1, buf2, buf3, zbuf, acc,
                 g0, g1, g2, g3, s0, s1, s2, s3):
    cid = lax.axis_index("c")
    sid = lax.axis_index("s")

    zv = jnp.zeros((16,), jnp.float32)

    def zb(i, carry):
        zbuf[i, pl.ds(0, 16)] = zv
        zbuf[i, pl.ds(16, 16)] = zv
        return carry

    lax.fori_loop(0, _ZR, zb, 0)

    zsems = (s0, s1, s2, s3)
    zd = [None] * _NZ
    for i in range(_NZ):
        if i >= 4:
            zd[i - 4].wait()
        zd[i] = pltpu.async_copy(
            zbuf, acc.at[pl.ds(sid * _RPS + i * _ZR, _ZR)], zsems[i % 4])
    for i in range(_NZ - 4, _NZ):
        zd[i].wait()

    @pl.when(sid == 0)
    def _():
        pltpu.sync_copy(zbuf.at[pl.ds(0, _REM)], acc.at[pl.ds(_REM0, _REM)])

    plsc.subcore_barrier()

    bufs = (buf0, buf1, buf2, buf3)
    gsems = (g0, g1, g2, g3)
    ssems = (s0, s1, s2, s3)
    extra = jnp.where(cid == 0, 2, 1)
    nb = 6 + jnp.where(sid < extra, 1, 0)
    core_base = cid * 98

    def sb_step(j, carry):
        row0 = (core_base + sid + _NS * j) * 32
        pltpu.sync_copy(src2_hbm.at[pl.ds(row0, 32)], sidx4)
        pltpu.sync_copy(dst2_hbm.at[pl.ds(row0, 32)], didx4)
        _edge_pipeline(y_hbm, acc, sidx4, didx4, bufs, gsems, ssems, 32)
        return carry

    lax.fori_loop(0, nb, sb_step, 0)

    @pl.when((cid == 1) & (sid == _NS - 1))
    def _():
        ntail = _CHUNKS - 195 * 32
        pltpu.sync_copy(src2_hbm.at[pl.ds(195 * 32, ntail)],
                        sidx4.at[pl.ds(0, ntail)])
        pltpu.sync_copy(dst2_hbm.at[pl.ds(195 * 32, ntail)],
                        didx4.at[pl.ds(0, ntail)])
        _edge_pipeline(y_hbm, acc, sidx4, didx4, bufs, gsems, ssems, ntail)

    plsc.subcore_barrier()

    fd = [None] * _NZ
    for i in range(_NZ):
        if i >= 4:
            fd[i - 4].wait()
        r0 = sid * _RPS + i * _ZR
        fd[i] = pltpu.async_copy(
            acc.at[pl.ds(r0, _ZR)],
            out_hbm.at[pl.ds(cid * N_NODES + r0, _ZR)], ssems[i % 4])
    for i in range(_NZ - 4, _NZ):
        fd[i].wait()

    @pl.when(sid == 0)
    def _():
        pltpu.sync_copy(acc.at[pl.ds(_REM0, _REM)],
                        out_hbm.at[pl.ds(cid * N_NODES + _REM0, _REM)])


def _deg_body(dst2_hbm, out_hbm, didx4, ones_buf, zbuf, acc, s0, s1, s2, s3):
    cid = lax.axis_index("c")
    sid = lax.axis_index("s")
    ssems = (s0, s1, s2, s3)

    zero = jnp.zeros((16,), jnp.float32)
    one = jnp.ones((16,), jnp.float32)

    def fb(i, carry):
        zbuf[i, pl.ds(0, 16)] = zero
        zbuf[i, pl.ds(16, 16)] = zero
        return carry

    lax.fori_loop(0, _ZR, fb, 0)

    def ob(i, carry):
        ones_buf[i, pl.ds(0, 16)] = one
        ones_buf[i, pl.ds(16, 16)] = one
        return carry

    lax.fori_loop(0, _CH, ob, 0)

    zd = [None] * _NZ
    for i in range(_NZ):
        if i >= 4:
            zd[i - 4].wait()
        zd[i] = pltpu.async_copy(
            zbuf, acc.at[pl.ds(sid * _RPS + i * _ZR, _ZR)], ssems[i % 4])
    for i in range(_NZ - 4, _NZ):
        zd[i].wait()

    @pl.when(sid == 0)
    def _():
        pltpu.sync_copy(zbuf.at[pl.ds(0, _REM)], acc.at[pl.ds(_REM0, _REM)])

    plsc.subcore_barrier()

    extra = jnp.where(cid == 0, 2, 1)
    nb = 6 + jnp.where(sid < extra, 1, 0)
    core_base = cid * 98

    def scatter_run(n):
        sd = [None] * n
        for t in range(n):
            if t >= 8:
                sd[t - 8].wait()
            sd[t] = pltpu.async_copy(ones_buf, acc.at[didx4.at[t]],
                                     ssems[t % 4], add=True)
        for t in range(max(0, n - 8), n):
            sd[t].wait()

    def sb_step(j, carry):
        row0 = (core_base + sid + _NS * j) * 32
        pltpu.sync_copy(dst2_hbm.at[pl.ds(row0, 32)], didx4)
        scatter_run(32)
        return carry

    lax.fori_loop(0, nb, sb_step, 0)

    @pl.when((cid == 1) & (sid == _NS - 1))
    def _():
        ntail = _CHUNKS - 195 * 32
        pltpu.sync_copy(dst2_hbm.at[pl.ds(195 * 32, ntail)],
                        didx4.at[pl.ds(0, ntail)])
        scatter_run(ntail)

    plsc.subcore_barrier()

    fd = [None] * _NZ
    for i in range(_NZ):
        if i >= 4:
            fd[i - 4].wait()
        r0 = sid * _RPS + i * _ZR
        fd[i] = pltpu.async_copy(
            acc.at[pl.ds(r0, _ZR)],
            out_hbm.at[pl.ds(cid * N_NODES + r0, _ZR)], ssems[i % 4])
    for i in range(_NZ - 4, _NZ):
        fd[i].wait()

    @pl.when(sid == 0)
    def _():
        pltpu.sync_copy(acc.at[pl.ds(_REM0, _REM)],
                        out_hbm.at[pl.ds(cid * N_NODES + _REM0, _REM)])


_deg_call = pl.kernel(
    _deg_body,
    out_type=jax.ShapeDtypeStruct((_NC * N_NODES, DIM), jnp.float32),
    mesh=_sc_mesh,
    scratch_types=[
        pltpu.VMEM((32, _CH), jnp.int32),
        pltpu.VMEM((_CH, DIM), jnp.float32),
        pltpu.VMEM((_ZR, DIM), jnp.float32),
        pltpu.VMEM_SHARED((N_NODES, DIM), jnp.float32),
        pltpu.SemaphoreType.DMA,
        pltpu.SemaphoreType.DMA,
        pltpu.SemaphoreType.DMA,
        pltpu.SemaphoreType.DMA,
    ],
    compiler_params=pltpu.CompilerParams(use_tc_tiling_on_sc=False),
)


_segsum_call = pl.kernel(
    _segsum_body,
    out_type=jax.ShapeDtypeStruct((_NC * N_NODES, DIM), jnp.float32),
    mesh=_sc_mesh,
    scratch_types=[
        pltpu.VMEM((32, _CH), jnp.int32),
        pltpu.VMEM((32, _CH), jnp.int32),
        pltpu.VMEM((_CH, DIM), jnp.float32),
        pltpu.VMEM((_CH, DIM), jnp.float32),
        pltpu.VMEM((_CH, DIM), jnp.float32),
        pltpu.VMEM((_CH, DIM), jnp.float32),
        pltpu.VMEM((_ZR, DIM), jnp.float32),
        pltpu.VMEM_SHARED((N_NODES, DIM), jnp.float32),
        pltpu.SemaphoreType.DMA,
        pltpu.SemaphoreType.DMA,
        pltpu.SemaphoreType.DMA,
        pltpu.SemaphoreType.DMA,
        pltpu.SemaphoreType.DMA,
        pltpu.SemaphoreType.DMA,
        pltpu.SemaphoreType.DMA,
        pltpu.SemaphoreType.DMA,
    ],
    compiler_params=pltpu.CompilerParams(use_tc_tiling_on_sc=False),
)


_BLK = 5000
_NBLK = N_NODES // _BLK


def _matmul_body(x_ref, w_ref, o_ref):
    o_ref[...] = jnp.dot(x_ref[...], w_ref[...],
                         preferred_element_type=jnp.float32)


def _node_matmul(x, w):
    fin = x.shape[1]
    return pl.pallas_call(
        _matmul_body,
        grid=(_NBLK,),
        in_specs=[pl.BlockSpec((_BLK, fin), lambda i: (i, 0)),
                  pl.BlockSpec((fin, DIM), lambda i: (0, 0))],
        out_specs=pl.BlockSpec((_BLK, DIM), lambda i: (i, 0)),
        out_shape=jax.ShapeDtypeStruct((N_NODES, DIM), jnp.float32),
    )(x, w)


def _stats_accum(m, sum_ref, ssq_ref):
    ones = jnp.ones((8, m.shape[0]), jnp.float32)
    ps = jnp.dot(ones, m, preferred_element_type=jnp.float32)
    pq = jnp.dot(ones, m * m, preferred_element_type=jnp.float32)

    @pl.when(pl.program_id(0) == 0)
    def _():
        sum_ref[...] = ps
        ssq_ref[...] = pq

    @pl.when(pl.program_id(0) != 0)
    def _():
        sum_ref[...] += ps
        ssq_ref[...] += pq


def _gin0_body(y_ref, s0_ref, s1_ref, b1_ref, w2_ref, b2_ref,
               m_ref, sum_ref, ssq_ref):
    u = jnp.maximum(y_ref[...] + s0_ref[...] + s1_ref[...] + b1_ref[...], 0.0)
    m = jnp.maximum(
        jnp.dot(u, w2_ref[...], preferred_element_type=jnp.float32)
        + b2_ref[...], 0.0)
    m_ref[...] = m
    _stats_accum(m, sum_ref, ssq_ref)


def _gin_body(m_ref, s0_ref, s1_ref, d0_ref, d1_ref, sum_ref, ssq_ref,
              g_ref, b_ref, w1_ref, b1_ref, w2_ref, b2_ref,
              mo_ref, sumo_ref, ssqo_ref):
    a, c = _bn_scale(sum_ref[...], ssq_ref[...], g_ref[...], b_ref[...])
    t = (a * (m_ref[...] + s0_ref[...] + s1_ref[...])
         + c * (1.0 + d0_ref[...] + d1_ref[...]))
    u = jnp.maximum(
        jnp.dot(t, w1_ref[...], preferred_element_type=jnp.float32)
        + b1_ref[...], 0.0)
    m = jnp.maximum(
        jnp.dot(u, w2_ref[...], preferred_element_type=jnp.float32)
        + b2_ref[...], 0.0)
    mo_ref[...] = m
    _stats_accum(m, sumo_ref, ssqo_ref)


_nodes_spec = pl.BlockSpec((_BLK, DIM), lambda i: (i, 0))
_part0_spec = pl.BlockSpec((_BLK, DIM), lambda i: (i, 0))
_part1_spec = pl.BlockSpec((_BLK, DIM), lambda i: (i + _NBLK, 0))
_w_spec = pl.BlockSpec((DIM, DIM), lambda i: (0, 0))
_b_spec = pl.BlockSpec((1, DIM), lambda i: (0, 0))
_stat_spec = pl.BlockSpec((8, DIM), lambda i: (0, 0))
_mlp_out_shapes = (
    jax.ShapeDtypeStruct((N_NODES, DIM), jnp.float32),
    jax.ShapeDtypeStruct((8, DIM), jnp.float32),
    jax.ShapeDtypeStruct((8, DIM), jnp.float32),
)


def _gin_mlp0(y, s, b1, w2, b2):
    return pl.pallas_call(
        _gin0_body,
        grid=(_NBLK,),
        in_specs=[_nodes_spec, _part0_spec, _part1_spec,
                  _b_spec, _w_spec, _b_spec],
        out_specs=(_nodes_spec, _stat_spec, _stat_spec),
        out_shape=_mlp_out_shapes,
    )(y, s, s, b1, w2, b2)


def _gin_mlp(m, s, d, s8, q8, g, b, w1, b1, w2, b2):
    return pl.pallas_call(
        _gin_body,
        grid=(_NBLK,),
        in_specs=[_nodes_spec, _part0_spec, _part1_spec,
                  _part0_spec, _part1_spec,
                  _stat_spec, _stat_spec, _b_spec, _b_spec,
                  _w_spec, _b_spec, _w_spec, _b_spec],
        out_specs=(_nodes_spec, _stat_spec, _stat_spec),
        out_shape=_mlp_out_shapes,
    )(m, s, s, d, d, s8, q8, g, b, w1, b1, w2, b2)


def _bn_scale(sum_v, ssq_v, g_v, b_v):
    s = jnp.sum(sum_v, axis=0, keepdims=True)
    q = jnp.sum(ssq_v, axis=0, keepdims=True)
    mu = s * (1.0 / N_NODES)
    var = q * (1.0 / N_NODES) - mu * mu
    a = g_v * lax.rsqrt(var + 1e-5)
    c = b_v - mu * a
    return a, c


def _bn_pool_body(m_ref, sum_ref, ssq_ref, g_ref, b_ref, bt_ref, p_ref):
    a, c = _bn_scale(sum_ref[...], ssq_ref[...], g_ref[...], b_ref[...])
    h = m_ref[...] * a + c
    brow = bt_ref[...].reshape(1, _BLK)
    oh = (lax.broadcasted_iota(jnp.int32, (N_GRAPHS, _BLK), 0)
          == brow).astype(jnp.float32)
    pp = jnp.dot(oh, h, preferred_element_type=jnp.float32)

    @pl.when(pl.program_id(0) == 0)
    def _():
        p_ref[...] = pp

    @pl.when(pl.program_id(0) != 0)
    def _():
        p_ref[...] += pp


def _bn_pool(m, s8, q8, g, b, batch3):
    return pl.pallas_call(
        _bn_pool_body,
        grid=(_NBLK,),
        in_specs=[_nodes_spec, _stat_spec, _stat_spec, _b_spec, _b_spec,
                  pl.BlockSpec((1, 1, _BLK), lambda i: (i, 0, 0))],
        out_specs=pl.BlockSpec((N_GRAPHS, DIM), lambda i: (0, 0)),
        out_shape=jax.ShapeDtypeStruct((N_GRAPHS, DIM), jnp.float32),
    )(m, s8, q8, g, b, batch3)


_HG = 8


def _heads_body(t2_ref, t1_ref, w2s_ref, a2t_ref, b2c_ref, w1s_ref, b1c_ref,
                c2_ref, c1_ref):
    t2 = t2_ref[...].reshape(_HG, 1000)
    t1 = t1_ref[...]
    for g in range(_HG):
        t2g = t2[g:g + 1, :]
        oh = (lax.broadcasted_iota(jnp.int32, (32, 1000), 0)
              == t2g).astype(jnp.float32)
        c2 = jnp.zeros((32, 121), jnp.float32)
        for k in range(8):
            bk = lax.dot_general(w2s_ref[k], oh, (((1,), (1,)), ((), ())),
                                 preferred_element_type=jnp.float32)
            c2 = c2 + jnp.dot(bk, a2t_ref[k],
                              preferred_element_type=jnp.float32)
        c2_ref[g] = c2 + b2c_ref[...]

        t1g = t1[g]
        c1 = jnp.zeros((32, 17), jnp.float32)
        for k in range(8):
            c1 = c1 + jnp.dot(w1s_ref[k], t1g[:, k:k + 17],
                              preferred_element_type=jnp.float32)
        c1_ref[g] = c1 + b1c_ref[...]


def _conv_heads(t2r, t1, w2s, a2t, b2c, w1s, b1c):
    return pl.pallas_call(
        _heads_body,
        grid=(N_GRAPHS // _HG,),
        in_specs=[
            pl.BlockSpec((_HG, 1, 1000), lambda i: (i, 0, 0)),
            pl.BlockSpec((_HG, 20, 24), lambda i: (i, 0, 0)),
            pl.BlockSpec((8, 32, 1000), lambda i: (0, 0, 0)),
            pl.BlockSpec((8, 32, 121), lambda i: (0, 0, 0)),
            pl.BlockSpec((32, 1), lambda i: (0, 0)),
            pl.BlockSpec((8, 32, 20), lambda i: (0, 0, 0)),
            pl.BlockSpec((32, 1), lambda i: (0, 0)),
        ],
        out_specs=(pl.BlockSpec((_HG, 32, 121), lambda i: (i, 0, 0)),
                   pl.BlockSpec((_HG, 32, 17), lambda i: (i, 0, 0))),
        out_shape=(jax.ShapeDtypeStruct((N_GRAPHS, 32, 121), jnp.float32),
                   jax.ShapeDtypeStruct((N_GRAPHS, 32, 17), jnp.float32)),
    )(t2r, t1, w2s, a2t, b2c, w1s, b1c)


def _pool_final_body(m_ref, sum_ref, ssq_ref, g_ref, b_ref, bt_ref,
                     c1f_ref, c2f_ref, wxd_ref, bxd_ref, w1x_ref, b1x_ref,
                     w2x_ref, b2x_ref, wa_ref, wb_ref, wc_ref, bf1_ref,
                     wf2_ref, bf2_ref, wo_ref, bo_ref, o_ref, pacc):
    i = pl.program_id(0)
    a, c = _bn_scale(sum_ref[...], ssq_ref[...], g_ref[...], b_ref[...])
    h = m_ref[...] * a + c
    brow = bt_ref[...].reshape(1, _BLK)
    oh = (lax.broadcasted_iota(jnp.int32, (N_GRAPHS, _BLK), 0)
          == brow).astype(jnp.float32)
    pp = jnp.dot(oh, h, preferred_element_type=jnp.float32)

    @pl.when(i == 0)
    def _():
        pacc[...] = pp

    @pl.when(i != 0)
    def _():
        pacc[...] += pp

    @pl.when(i == _NBLK - 1)
    def _():
        _final_compute(pacc[...], c1f_ref, c2f_ref, wxd_ref, bxd_ref,
                       w1x_ref, b1x_ref, w2x_ref, b2x_ref, wa_ref, wb_ref,
                       wc_ref, bf1_ref, wf2_ref, bf2_ref, wo_ref, bo_ref,
                       o_ref)


def _pool_final(m, s8, q8, g, b, batch3, c1f, c2f, *weights):
    wspecs = [pl.BlockSpec(w.shape, lambda i: (0, 0)) for w in weights]
    return pl.pallas_call(
        _pool_final_body,
        grid=(_NBLK,),
        in_specs=[_nodes_spec, _stat_spec, _stat_spec, _b_spec, _b_spec,
                  pl.BlockSpec((1, 1, _BLK), lambda i: (i, 0, 0)),
                  pl.BlockSpec(c1f.shape, lambda i: (0, 0)),
                  pl.BlockSpec(c2f.shape, lambda i: (0, 0))] + wspecs,
        out_specs=pl.BlockSpec((N_GRAPHS, 1), lambda i: (0, 0)),
        out_shape=jax.ShapeDtypeStruct((N_GRAPHS, 1), jnp.float32),
        scratch_shapes=[pltpu.VMEM((N_GRAPHS, DIM), jnp.float32)],
    )(m, s8, q8, g, b, batch3, c1f, c2f, *weights)


def _final_compute(pooled, c1f_ref, c2f_ref, wxd_ref, bxd_ref, w1x_ref,
                   b1x_ref, w2x_ref, b2x_ref, wa_ref, wb_ref, wc_ref,
                   bf1_ref, wf2_ref, bf2_ref, wo_ref, bo_ref, o_ref):
    f32 = jnp.float32
    xd = jnp.maximum(
        jnp.dot(pooled, wxd_ref[...], preferred_element_type=f32)
        + bxd_ref[...], 0.0)
    xt1 = jnp.dot(c1f_ref[...], w1x_ref[...],
                  preferred_element_type=f32) + b1x_ref[...]
    xt2 = jnp.dot(c2f_ref[...], w2x_ref[...],
                  preferred_element_type=f32) + b2x_ref[...]
    z = jnp.maximum(
        jnp.dot(xd, wa_ref[...], preferred_element_type=f32)
        + jnp.dot(xt1, wb_ref[...], preferred_element_type=f32)
        + jnp.dot(xt2, wc_ref[...], preferred_element_type=f32)
        + bf1_ref[...], 0.0)
    z2 = jnp.maximum(
        jnp.dot(z, wf2_ref[...], preferred_element_type=f32)
        + bf2_ref[...], 0.0)
    o_ref[...] = jnp.dot(z2, wo_ref[...],
                         preferred_element_type=f32) + bo_ref[...]


def kernel(x, edge_index, batch, target1, target2, params):
    p = params
    src2 = edge_index[0].reshape(_CHUNKS, _CH)
    dst2 = edge_index[1].reshape(_CHUNKS, _CH)

    def row(v):
        return v.reshape(1, DIM)

    batch3 = batch.astype(jnp.int32).reshape(_NBLK, 1, _BLK)
    deg = _deg_call(dst2)
    y = _node_matmul(x, p['gin0_w1'])
    s = _segsum_call(y, src2, dst2)
    m, s8, q8 = _gin_mlp0(y, s, row(p['gin0_b1']),
                          p['gin0_w2'], row(p['gin0_b2']))
    for i in range(1, 5):
        s = _segsum_call(m, src2, dst2)
        m, s8, q8 = _gin_mlp(m, s, deg, s8, q8,
                             row(p['bn%d_g' % (i - 1)]),
                             row(p['bn%d_b' % (i - 1)]),
                             p['gin%d_w1' % i], row(p['gin%d_b1' % i]),
                             p['gin%d_w2' % i], row(p['gin%d_b2' % i]))
    t2r = target2.astype(jnp.int32).reshape(N_GRAPHS, 1, 1000)
    epad = jnp.concatenate(
        [p['emb'], jnp.zeros((6, 128), jnp.float32)], axis=0)
    a2t = jnp.stack([epad[:, k:k + 121] for k in range(8)])
    w2s = jnp.transpose(p['cxt2_w'], (2, 0, 1))
    w1s = jnp.transpose(p['cxt1_w'], (2, 0, 1))
    c2, c1 = _conv_heads(t2r, target1, w2s, a2t,
                         p['cxt2_b'].reshape(32, 1), w1s,
                         p['cxt1_b'].reshape(32, 1))
    c2f = c2.reshape(N_GRAPHS, 32 * 121)
    c1f = c1.reshape(N_GRAPHS, 32 * 17)

    fw = p['fc1_w']
    out = _pool_final(
        m, s8, q8, row(p['bn4_g']), row(p['bn4_b']), batch3, c1f, c2f,
        p['fc1_xd_w'], p['fc1_xd_b'].reshape(1, -1),
        p['fc1_xt_w'], p['fc1_xt_b'].reshape(1, -1),
        p['fc2_xt_w'], p['fc2_xt_b'].reshape(1, -1),
        fw[0:128], fw[128:256], fw[256:384], p['fc1_b'].reshape(1, -1),
        p['fc2_w'], p['fc2_b'].reshape(1, -1),
        p['out_w'], p['out_b'].reshape(1, -1))
    return out
